# baseline (speedup 1.0000x reference)
import functools

import jax
import jax.numpy as jnp
from jax import lax
from jax.experimental import pallas as pl
from jax.experimental.pallas import tpu as pltpu
from jax.experimental.pallas import tpu_sc as plsc

N = 10000
E = 320000
IN_CH = 128
HID = 32
OUT_CH = 128
HEADS = 8

_BLK = 1000
_EB = 64
_NB = 40
_NCHUNK = N // _NB

_IFILL = 512
_EPAD = E + _IFILL

_ET1 = E // 16
_NF1 = _ET1 // _EB
_P1 = _NF1 // 2
_TL1 = _ET1 - _NF1 * _EB

_EB2 = 128
_ET2 = E // 32
_NF2 = _ET2 // _EB2
_P2 = _NF2 // 2
_TL2 = _ET2 - _NF2 * _EB2
_TC2 = (_NF2 * _EB2) % _IFILL

_G1W = 144
_G2W = 48


def _dense1_body(x_ref, w1_ref, b1_ref, wc1_ref, s1_ref, d1_ref,
                 g_ref, ad_ref, mxs_ref, mxd_ref):
    i = pl.program_id(0)
    h0 = jax.nn.relu(
        jnp.dot(x_ref[...], w1_ref[...], preferred_element_type=jnp.float32)
        + b1_ref[...][None, :])
    h1 = jnp.dot(h0, wc1_ref[...], preferred_element_type=jnp.float32)
    a1s = jnp.dot(h1, s1_ref[...], preferred_element_type=jnp.float32)
    a1d = jnp.dot(h1, d1_ref[...], preferred_element_type=jnp.float32)
    z12 = jnp.zeros((_BLK, 12), jnp.float32)
    g_ref[0] = jnp.concatenate([a1s[:, 0:4], z12, h1[:, 0:128]], axis=1)
    g_ref[1] = jnp.concatenate([a1s[:, 4:8], z12, h1[:, 128:256]], axis=1)
    ad_ref[0] = jnp.concatenate([a1d[:, 0:4], z12], axis=1)
    ad_ref[1] = jnp.concatenate([a1d[:, 4:8], z12], axis=1)
    bs = jnp.max(a1s, axis=0, keepdims=True)
    bd = jnp.max(a1d, axis=0, keepdims=True)

    @pl.when(i == 0)
    def _():
        mxs_ref[...] = bs
        mxd_ref[...] = bd

    @pl.when(i > 0)
    def _():
        mxs_ref[...] = jnp.maximum(mxs_ref[...], bs)
        mxd_ref[...] = jnp.maximum(mxd_ref[...], bd)


def _dense1(x, w1, b1, wc1, s1, d1):
    return pl.pallas_call(
        _dense1_body,
        grid=(N // _BLK,),
        in_specs=[
            pl.BlockSpec((_BLK, IN_CH), lambda i: (i, 0)),
            pl.BlockSpec((IN_CH, HID), lambda i: (0, 0)),
            pl.BlockSpec((HID,), lambda i: (0,)),
            pl.BlockSpec((HID, HEADS * HID), lambda i: (0, 0)),
            pl.BlockSpec((HEADS * HID, HEADS), lambda i: (0, 0)),
            pl.BlockSpec((HEADS * HID, HEADS), lambda i: (0, 0)),
        ],
        out_specs=[
            pl.BlockSpec((2, _BLK, _G1W), lambda i: (0, i, 0)),
            pl.BlockSpec((2, _BLK, 16), lambda i: (0, i, 0)),
            pl.BlockSpec((1, HEADS), lambda i: (0, 0)),
            pl.BlockSpec((1, HEADS), lambda i: (0, 0)),
        ],
        out_shape=[
            jax.ShapeDtypeStruct((2, N, _G1W), jnp.float32),
            jax.ShapeDtypeStruct((2, N, 16), jnp.float32),
            jax.ShapeDtypeStruct((1, HEADS), jnp.float32),
            jax.ShapeDtypeStruct((1, HEADS), jnp.float32),
        ],
    )(x, w1, b1, wc1, s1, d1)


def _sc1_body(edge_ref, g_ref, adt_ref, cvec_ref, out_ref,
              acc, cbuf,
              rb0, rb1, ab0, ab1, sb0, sb1,
              eidx, si0, si1, di0, di1, dS0, dS1, dSt,
              sg0, sg1, sa0, sa1, ss0, ss1):
    cid = lax.axis_index("c")
    sid = lax.axis_index("s")
    coff = cid * N
    pltpu.sync_copy(cvec_ref, cbuf)
    cv = cbuf[cid]

    for j in range((_NCHUNK + 15) // 16):
        cidx = sid + 16 * j

        @pl.when(cidx < _NCHUNK)
        def _():
            rb = cidx * _NB
            pltpu.sync_copy(g_ref.at[pl.ds(coff + rb, _NB)],
                            rb0.at[pl.ds(0, _NB)])
            pltpu.sync_copy(adt_ref.at[pl.ds(coff + rb, _NB)],
                            ab0.at[pl.ds(0, _NB)])

            def _init_row(r, _):
                asv = rb0[r, pl.ds(0, 16)]
                adv = ab0[r, pl.ds(0, 16)]
                z = asv + adv
                al = jnp.where(z > 0, z, 0.2 * z) - cv
                exv = jnp.exp(al)
                rb0[r, pl.ds(0, 16)] = exv
                for h in range(4):
                    exh = exv[h]
                    for k in range(2):
                        c0 = 16 + 32 * h + 16 * k
                        rb0[r, pl.ds(c0, 16)] = exh * rb0[r, pl.ds(c0, 16)]
                return 0

            lax.fori_loop(0, _NB, _init_row, 0)
            pltpu.sync_copy(rb0.at[pl.ds(0, _NB)], acc.at[pl.ds(rb, _NB)])

    plsc.subcore_barrier()

    bufs = ((rb0, ab0, sb0, si0, di0, dS0, sg0, sa0, ss0),
            (rb1, ab1, sb1, si1, di1, dS1, sg1, sa1, ss1))
    tbase = sid * _ET1

    def _refill(q):
        pltpu.sync_copy(edge_ref.at[:, pl.ds(tbase + q * _IFILL, _IFILL)],
                        eidx)

    def _issue(i, bid):
        rbE, abE, sbuf, sidx, didx2, didxS, sg, sa, ss = bufs[i]
        m = 64 * lax.rem(bid, _IFILL // _EB)
        for k in range(_EB // 16):
            sl = pl.ds(16 * k, 16)
            sidx[sl] = eidx[0, pl.ds(m + 16 * k, 16)] + coff
            didx2[sl] = eidx[1, pl.ds(m + 16 * k, 16)] + coff
        pltpu.async_copy(g_ref.at[sidx], rbE, sg)
        pltpu.async_copy(adt_ref.at[didx2], abE, sa)

    def _wait_gather(i):
        rbE, abE, sbuf, sidx, didx2, didxS, sg, sa, ss = bufs[i]
        pltpu.make_async_copy(g_ref.at[sidx], rbE, sg).wait()
        pltpu.make_async_copy(adt_ref.at[didx2], abE, sa).wait()

    def _wait_scatter(i):
        rbE, abE, sbuf, sidx, didx2, didxS, sg, sa, ss = bufs[i]
        pltpu.make_async_copy(sbuf, acc.at[didxS], ss).wait()

    def _compute_scatter(i):
        rbE, abE, sbuf, sidx, didx2, didxS, sg, sa, ss = bufs[i]
        for k in range(_EB // 16):
            sl = pl.ds(16 * k, 16)
            didxS[sl] = didx2[sl] - coff

        def _edge(e, _):
            asv = rbE[e, pl.ds(0, 16)]
            adv = abE[e, pl.ds(0, 16)]
            z = asv + adv
            al = jnp.where(z > 0, z, 0.2 * z) - cv
            exv = jnp.exp(al)
            sbuf[e, pl.ds(0, 16)] = exv
            for h in range(4):
                exh = exv[h]
                for k in range(2):
                    c0 = 16 + 32 * h + 16 * k
                    sbuf[e, pl.ds(c0, 16)] = exh * rbE[e, pl.ds(c0, 16)]
            return 0

        lax.fori_loop(0, _EB, _edge, 0, unroll=2)
        pltpu.async_copy(sbuf, acc.at[didxS], ss, add=True)

    _refill(0)
    _issue(0, 0)
    _issue(1, 1)

    def _pair(p, _):
        @pl.when(lax.rem(p + 1, (_IFILL // _EB) // 2) == 0)
        def _():
            _refill((p + 1) // ((_IFILL // _EB) // 2))

        _wait_gather(0)

        @pl.when(p > 0)
        def _():
            _wait_scatter(0)

        _compute_scatter(0)

        @pl.when(2 * p + 2 < _NF1)
        def _():
            _issue(0, 2 * p + 2)

        _wait_gather(1)

        @pl.when(p > 0)
        def _():
            _wait_scatter(1)

        _compute_scatter(1)

        @pl.when(2 * p + 3 < _NF1)
        def _():
            _issue(1, 2 * p + 3)

        return 0

    lax.fori_loop(0, _P1, _pair, 0)
    _wait_scatter(0)
    _wait_scatter(1)

    for k in range(_TL1 // 16):
        sl = pl.ds(16 * k, 16)
        si0[sl] = eidx[0, pl.ds(16 * k, 16)] + coff
        di0[sl] = eidx[1, pl.ds(16 * k, 16)] + coff
        dSt[sl] = eidx[1, pl.ds(16 * k, 16)]
    pltpu.async_copy(g_ref.at[si0.at[pl.ds(0, _TL1)]],
                     rb0.at[pl.ds(0, _TL1)], sg0)
    pltpu.async_copy(adt_ref.at[di0.at[pl.ds(0, _TL1)]],
                     ab0.at[pl.ds(0, _TL1)], sa0)
    pltpu.make_async_copy(g_ref.at[si0.at[pl.ds(0, _TL1)]],
                          rb0.at[pl.ds(0, _TL1)], sg0).wait()
    pltpu.make_async_copy(adt_ref.at[di0.at[pl.ds(0, _TL1)]],
                          ab0.at[pl.ds(0, _TL1)], sa0).wait()

    def _tail_edge(e, _):
        asv = rb0[e, pl.ds(0, 16)]
        adv = ab0[e, pl.ds(0, 16)]
        z = asv + adv
        al = jnp.where(z > 0, z, 0.2 * z) - cv
        exv = jnp.exp(al)
        sb0[e, pl.ds(0, 16)] = exv
        for h in range(4):
            exh = exv[h]
            for k in range(2):
                c0 = 16 + 32 * h + 16 * k
                sb0[e, pl.ds(c0, 16)] = exh * rb0[e, pl.ds(c0, 16)]
        return 0

    lax.fori_loop(0, _TL1, _tail_edge, 0)
    pltpu.async_copy(sb0.at[pl.ds(0, _TL1)],
                     acc.at[dSt], ss0, add=True)
    pltpu.make_async_copy(sb0.at[pl.ds(0, _TL1)],
                          acc.at[dSt], ss0).wait()
    plsc.subcore_barrier()

    for j in range((_NCHUNK + 15) // 16):
        cidx = sid + 16 * j

        @pl.when(cidx < _NCHUNK)
        def _():
            rb = cidx * _NB
            pltpu.sync_copy(acc.at[pl.ds(rb, _NB)], rb0.at[pl.ds(0, _NB)])
            pltpu.sync_copy(rb0.at[pl.ds(0, _NB)],
                            out_ref.at[cid, pl.ds(rb, _NB)])


def _sc1(edge_index, g, adt, cvec):
    mesh = plsc.VectorSubcoreMesh(core_axis_name="c", subcore_axis_name="s")
    f = pl.kernel(
        _sc1_body,
        out_type=jax.ShapeDtypeStruct((2, N, _G1W), jnp.float32),
        mesh=mesh,
        scratch_types=[
            pltpu.VMEM_SHARED((N, _G1W), jnp.float32),
            pltpu.VMEM((2, 16), jnp.float32),
            pltpu.VMEM((_EB, _G1W), jnp.float32),
            pltpu.VMEM((_EB, _G1W), jnp.float32),
            pltpu.VMEM((_EB, 16), jnp.float32),
            pltpu.VMEM((_EB, 16), jnp.float32),
            pltpu.VMEM((_EB, _G1W), jnp.float32),
            pltpu.VMEM((_EB, _G1W), jnp.float32),
            pltpu.VMEM((2, _IFILL), jnp.int32),
            pltpu.VMEM((_EB,), jnp.int32),
            pltpu.VMEM((_EB,), jnp.int32),
            pltpu.VMEM((_EB,), jnp.int32),
            pltpu.VMEM((_EB,), jnp.int32),
            pltpu.VMEM((_EB,), jnp.int32),
            pltpu.VMEM((_EB,), jnp.int32),
            pltpu.VMEM((_TL1,), jnp.int32),
            pltpu.SemaphoreType.DMA,
            pltpu.SemaphoreType.DMA,
            pltpu.SemaphoreType.DMA,
            pltpu.SemaphoreType.DMA,
            pltpu.SemaphoreType.DMA,
            pltpu.SemaphoreType.DMA,
        ],
        compiler_params=pltpu.CompilerParams(use_tc_tiling_on_sc=False),
    )
    return f(edge_index, g, adt, cvec)


def _dense2_body(acc_ref, bc1_ref, wc2_ref, s2_ref, d2_ref,
                 t2_ref, ad2_ref, mxs_ref, mxd_ref):
    i = pl.program_id(0)
    a0 = acc_ref[0]
    a1 = acc_ref[1]
    msg = jnp.concatenate([a0[:, 16:_G1W], a1[:, 16:_G1W]], axis=1)
    dens = []
    for c in range(2):
        a = a0 if c == 0 else a1
        for h in range(4):
            dens.append(jnp.broadcast_to(a[:, h:h + 1], (_BLK, HID)))
    den = jnp.concatenate(dens, axis=1)
    g1 = msg / (den + 1e-16) + bc1_ref[...][None, :]
    g1 = jnp.where(g1 > 0, g1, jnp.exp(g1) - 1.0)
    h2 = jnp.dot(g1, wc2_ref[...], preferred_element_type=jnp.float32)
    a2s = jnp.dot(h2, s2_ref[...], preferred_element_type=jnp.float32)
    a2d = jnp.dot(h2, d2_ref[...], preferred_element_type=jnp.float32)
    z15 = jnp.zeros((_BLK, 15), jnp.float32)
    t2_ref[...] = jnp.concatenate([a2s[:, 0:1], z15, h2], axis=1)
    ad2_ref[...] = jnp.concatenate([a2d[:, 0:1], z15], axis=1)
    bs = jnp.max(a2s, axis=0, keepdims=True)
    bd = jnp.max(a2d, axis=0, keepdims=True)

    @pl.when(i == 0)
    def _():
        mxs_ref[...] = bs
        mxd_ref[...] = bd

    @pl.when(i > 0)
    def _():
        mxs_ref[...] = jnp.maximum(mxs_ref[...], bs)
        mxd_ref[...] = jnp.maximum(mxd_ref[...], bd)


def _dense2(acc1, bc1, wc2, s2, d2):
    return pl.pallas_call(
        _dense2_body,
        grid=(N // _BLK,),
        in_specs=[
            pl.BlockSpec((2, _BLK, _G1W), lambda i: (0, i, 0)),
            pl.BlockSpec((HEADS * HID,), lambda i: (0,)),
            pl.BlockSpec((HEADS * HID, HID), lambda i: (0, 0)),
            pl.BlockSpec((HID, 8), lambda i: (0, 0)),
            pl.BlockSpec((HID, 8), lambda i: (0, 0)),
        ],
        out_specs=[
            pl.BlockSpec((_BLK, _G2W), lambda i: (i, 0)),
            pl.BlockSpec((_BLK, 16), lambda i: (i, 0)),
            pl.BlockSpec((1, 8), lambda i: (0, 0)),
            pl.BlockSpec((1, 8), lambda i: (0, 0)),
        ],
        out_shape=[
            jax.ShapeDtypeStruct((N, _G2W), jnp.float32),
            jax.ShapeDtypeStruct((N, 16), jnp.float32),
            jax.ShapeDtypeStruct((1, 8), jnp.float32),
            jax.ShapeDtypeStruct((1, 8), jnp.float32),
        ],
    )(acc1, bc1, wc2, s2, d2)


def _sc2_body(edge_ref, t2_ref, ad2_ref, cvec_ref, out_ref,
              acc, cbuf,
              rb0, rb1, ab0, ab1, sb0, sb1,
              eidx, si0, si1, di0, di1, dS0, dS1, dSt,
              sg0, sg1, sa0, sa1, ss0, ss1):
    cid = lax.axis_index("c")
    sid = lax.axis_index("s")
    pltpu.sync_copy(cvec_ref, cbuf)
    cv = cbuf[...]
    scale = jnp.where(cid == 0, 1.0, 0.0)

    for j in range((_NCHUNK + 15) // 16):
        cidx = sid + 16 * j

        @pl.when(cidx < _NCHUNK)
        def _():
            rb = cidx * _NB
            pltpu.sync_copy(t2_ref.at[pl.ds(rb, _NB)], rb0.at[pl.ds(0, _NB)])
            pltpu.sync_copy(ad2_ref.at[pl.ds(rb, _NB)], ab0.at[pl.ds(0, _NB)])

            def _init_row(r, _):
                asv = rb0[r, pl.ds(0, 16)]
                adv = ab0[r, pl.ds(0, 16)]
                z = asv + adv
                al = jnp.where(z > 0, z, 0.2 * z) - cv
                exv = jnp.exp(al) * scale
                rb0[r, pl.ds(0, 16)] = exv
                ex0 = exv[0]
                for k in range(2):
                    c0 = 16 + 16 * k
                    rb0[r, pl.ds(c0, 16)] = ex0 * rb0[r, pl.ds(c0, 16)]
                return 0

            lax.fori_loop(0, _NB, _init_row, 0)
            pltpu.sync_copy(rb0.at[pl.ds(0, _NB)], acc.at[pl.ds(rb, _NB)])

    plsc.subcore_barrier()

    bufs = ((rb0, ab0, sb0, si0, di0, dS0, sg0, sa0, ss0),
            (rb1, ab1, sb1, si1, di1, dS1, sg1, sa1, ss1))
    tbase = cid * (E // 2) + sid * _ET2

    def _refill(q):
        pltpu.sync_copy(edge_ref.at[:, pl.ds(tbase + q * _IFILL, _IFILL)],
                        eidx)

    def _issue(i, bid):
        rbE, abE, sbuf, sidx, didx2, didxS, sg, sa, ss = bufs[i]
        m = _EB2 * lax.rem(bid, _IFILL // _EB2)
        for k in range(_EB2 // 16):
            sl = pl.ds(16 * k, 16)
            sidx[sl] = eidx[0, pl.ds(m + 16 * k, 16)]
            didx2[sl] = eidx[1, pl.ds(m + 16 * k, 16)]
        pltpu.async_copy(t2_ref.at[sidx], rbE, sg)
        pltpu.async_copy(ad2_ref.at[didx2], abE, sa)

    def _wait_gather(i):
        rbE, abE, sbuf, sidx, didx2, didxS, sg, sa, ss = bufs[i]
        pltpu.make_async_copy(t2_ref.at[sidx], rbE, sg).wait()
        pltpu.make_async_copy(ad2_ref.at[didx2], abE, sa).wait()

    def _wait_scatter(i):
        rbE, abE, sbuf, sidx, didx2, didxS, sg, sa, ss = bufs[i]
        pltpu.make_async_copy(sbuf, acc.at[didxS], ss).wait()

    def _compute_scatter(i):
        rbE, abE, sbuf, sidx, didx2, didxS, sg, sa, ss = bufs[i]
        for k in range(_EB2 // 16):
            sl = pl.ds(16 * k, 16)
            didxS[sl] = didx2[sl]

        def _edge(e, _):
            asv = rbE[e, pl.ds(0, 16)]
            adv = abE[e, pl.ds(0, 16)]
            z = asv + adv
            al = jnp.where(z > 0, z, 0.2 * z) - cv
            exv = jnp.exp(al)
            sbuf[e, pl.ds(0, 16)] = exv
            ex0 = exv[0]
            for k in range(2):
                c0 = 16 + 16 * k
                sbuf[e, pl.ds(c0, 16)] = ex0 * rbE[e, pl.ds(c0, 16)]
            return 0

        lax.fori_loop(0, _EB2, _edge, 0)
        pltpu.async_copy(sbuf, acc.at[didxS], ss, add=True)

    _refill(0)
    _issue(0, 0)
    _issue(1, 1)

    def _pair(p, _):
        @pl.when(lax.rem(p + 1, (_IFILL // _EB2) // 2) == 0)
        def _():
            _refill((p + 1) // ((_IFILL // _EB2) // 2))

        _wait_gather(0)

        @pl.when(p > 0)
        def _():
            _wait_scatter(0)

        _compute_scatter(0)

        @pl.when(2 * p + 2 < _NF2)
        def _():
            _issue(0, 2 * p + 2)

        _wait_gather(1)

        @pl.when(p > 0)
        def _():
            _wait_scatter(1)

        _compute_scatter(1)

        @pl.when(2 * p + 3 < _NF2)
        def _():
            _issue(1, 2 * p + 3)

        return 0

    lax.fori_loop(0, _P2, _pair, 0)
    _wait_scatter(0)
    _wait_scatter(1)

    for k in range(_TL2 // 16):
        sl = pl.ds(16 * k, 16)
        si0[sl] = eidx[0, pl.ds(_TC2 + 16 * k, 16)]
        dSt[sl] = eidx[1, pl.ds(_TC2 + 16 * k, 16)]
    pltpu.async_copy(t2_ref.at[si0.at[pl.ds(0, _TL2)]],
                     rb0.at[pl.ds(0, _TL2)], sg0)
    pltpu.async_copy(ad2_ref.at[dSt], ab0.at[pl.ds(0, _TL2)], sa0)
    pltpu.make_async_copy(t2_ref.at[si0.at[pl.ds(0, _TL2)]],
                          rb0.at[pl.ds(0, _TL2)], sg0).wait()
    pltpu.make_async_copy(ad2_ref.at[dSt], ab0.at[pl.ds(0, _TL2)],
                          sa0).wait()

    def _tail_edge(e, _):
        asv = rb0[e, pl.ds(0, 16)]
        adv = ab0[e, pl.ds(0, 16)]
        z = asv + adv
        al = jnp.where(z > 0, z, 0.2 * z) - cv
        exv = jnp.exp(al)
        sb0[e, pl.ds(0, 16)] = exv
        ex0 = exv[0]
        for k in range(2):
            c0 = 16 + 16 * k
            sb0[e, pl.ds(c0, 16)] = ex0 * rb0[e, pl.ds(c0, 16)]
        return 0

    lax.fori_loop(0, _TL2, _tail_edge, 0)
    pltpu.async_copy(sb0.at[pl.ds(0, _TL2)], acc.at[dSt], ss0, add=True)
    pltpu.make_async_copy(sb0.at[pl.ds(0, _TL2)], acc.at[dSt], ss0).wait()
    plsc.subcore_barrier()

    for j in range((_NCHUNK + 15) // 16):
        cidx = sid + 16 * j

        @pl.when(cidx < _NCHUNK)
        def _():
            rb = cidx * _NB
            pltpu.sync_copy(acc.at[pl.ds(rb, _NB)], rb0.at[pl.ds(0, _NB)])
            pltpu.sync_copy(rb0.at[pl.ds(0, _NB)],
                            out_ref.at[cid, pl.ds(rb, _NB)])


def _sc2(edge_index, t2, ad2, cvec):
    mesh = plsc.VectorSubcoreMesh(core_axis_name="c", subcore_axis_name="s")
    f = pl.kernel(
        _sc2_body,
        out_type=jax.ShapeDtypeStruct((2, N, _G2W), jnp.float32),
        mesh=mesh,
        scratch_types=[
            pltpu.VMEM_SHARED((N, _G2W), jnp.float32),
            pltpu.VMEM((16,), jnp.float32),
            pltpu.VMEM((_EB2, _G2W), jnp.float32),
            pltpu.VMEM((_EB2, _G2W), jnp.float32),
            pltpu.VMEM((_EB2, 16), jnp.float32),
            pltpu.VMEM((_EB2, 16), jnp.float32),
            pltpu.VMEM((_EB2, _G2W), jnp.float32),
            pltpu.VMEM((_EB2, _G2W), jnp.float32),
            pltpu.VMEM((2, _IFILL), jnp.int32),
            pltpu.VMEM((_EB2,), jnp.int32),
            pltpu.VMEM((_EB2,), jnp.int32),
            pltpu.VMEM((_EB2,), jnp.int32),
            pltpu.VMEM((_EB2,), jnp.int32),
            pltpu.VMEM((_EB2,), jnp.int32),
            pltpu.VMEM((_EB2,), jnp.int32),
            pltpu.VMEM((_TL2,), jnp.int32),
            pltpu.SemaphoreType.DMA,
            pltpu.SemaphoreType.DMA,
            pltpu.SemaphoreType.DMA,
            pltpu.SemaphoreType.DMA,
            pltpu.SemaphoreType.DMA,
            pltpu.SemaphoreType.DMA,
        ],
        compiler_params=pltpu.CompilerParams(use_tc_tiling_on_sc=False),
    )
    return f(edge_index, t2, ad2, cvec)


def _dense3_body(acc_ref, bc2_ref, w2_ref, b2_ref, out_ref):
    a0 = acc_ref[0]
    a1 = acc_ref[1]
    msg = a0[:, 16:_G2W] + a1[:, 16:_G2W]
    den = jnp.broadcast_to(a0[:, 0:1] + a1[:, 0:1], (_BLK, HID))
    g2 = msg / (den + 1e-16) + bc2_ref[...][None, :]
    out_ref[...] = (
        jnp.dot(g2, w2_ref[...], preferred_element_type=jnp.float32)
        + b2_ref[...][None, :])


def _dense3(acc2, bc2, w2, b2):
    return pl.pallas_call(
        _dense3_body,
        grid=(N // _BLK,),
        in_specs=[
            pl.BlockSpec((2, _BLK, _G2W), lambda i: (0, i, 0)),
            pl.BlockSpec((HID,), lambda i: (0,)),
            pl.BlockSpec((HID, OUT_CH), lambda i: (0, 0)),
            pl.BlockSpec((OUT_CH,), lambda i: (0,)),
        ],
        out_specs=pl.BlockSpec((_BLK, OUT_CH), lambda i: (i, 0)),
        out_shape=jax.ShapeDtypeStruct((N, OUT_CH), jnp.float32),
    )(acc2, bc2, w2, b2)


def kernel(x, edge_index, W_lin1, b_lin1, Wc1, bc1, attc1_s, attc1_d,
           Wc2, bc2, attc2_s, attc2_d, W_lin2, b_lin2):
    eye = jnp.eye(HEADS, dtype=jnp.float32)
    s1 = (eye[:, None, :] * attc1_s.T[None, :, :]).reshape(HEADS * HID, HEADS)
    d1 = (eye[:, None, :] * attc1_d.T[None, :, :]).reshape(HEADS * HID, HEADS)
    s2 = jnp.pad(attc2_s.T, ((0, 0), (0, 7)))
    d2 = jnp.pad(attc2_d.T, ((0, 0), (0, 7)))

    epad = jnp.pad(edge_index, ((0, 0), (0, _EPAD - E)))

    g, adt, mxs, mxd = _dense1(x, W_lin1, b_lin1, Wc1, s1, d1)
    c1 = jnp.maximum(mxs[0] + mxd[0], 0.0)
    cvec1 = jnp.zeros((2, 16), jnp.float32).at[:, :4].set(c1.reshape(2, 4))
    acc1 = _sc1(epad, g.reshape(2 * N, _G1W),
                adt.reshape(2 * N, 16), cvec1)

    t2, ad2, mxs2, mxd2 = _dense2(acc1, bc1, Wc2, s2, d2)
    c2 = jnp.maximum(mxs2[0, 0] + mxd2[0, 0], 0.0)
    cvec2 = jnp.zeros((16,), jnp.float32).at[0].set(c2)
    acc2 = _sc2(epad, t2, ad2, cvec2)

    return _dense3(acc2, bc2, W_lin2, b_lin2)


# speedup vs baseline: 1.6916x; 1.6916x over previous
import functools

import jax
import jax.numpy as jnp
from jax import lax
from jax.experimental import pallas as pl
from jax.experimental.pallas import tpu as pltpu
from jax.experimental.pallas import tpu_sc as plsc

N = 10000
E = 320000
IN_CH = 128
HID = 32
OUT_CH = 128
HEADS = 8

_BLK = 1000
_EB = 64
_NB = 40
_NCHUNK = N // _NB

_IFILL = 512
_EPAD = E + _IFILL

_ET1 = E // 16
_NF1 = _ET1 // _EB
_P1 = _NF1 // 2
_TL1 = _ET1 - _NF1 * _EB

_EB2 = 128
_ET2 = E // 32
_NF2 = _ET2 // _EB2
_P2 = _NF2 // 2
_TL2 = _ET2 - _NF2 * _EB2
_TC2 = (_NF2 * _EB2) % _IFILL

_G1W = 144
_G2W = 48


def _dense1_body(x_ref, w1_ref, b1_ref, wc1_ref, s1_ref, d1_ref,
                 g_ref, ad_ref, mxs_ref, mxd_ref):
    i = pl.program_id(0)
    h0 = jax.nn.relu(
        jnp.dot(x_ref[...], w1_ref[...], preferred_element_type=jnp.float32)
        + b1_ref[...][None, :])
    h1 = jnp.dot(h0, wc1_ref[...], preferred_element_type=jnp.float32)
    a1s = jnp.dot(h1, s1_ref[...], preferred_element_type=jnp.float32)
    a1d = jnp.dot(h1, d1_ref[...], preferred_element_type=jnp.float32)
    z12 = jnp.zeros((_BLK, 12), jnp.float32)
    g_ref[0] = jnp.concatenate([a1s[:, 0:4], z12, h1[:, 0:128]], axis=1)
    g_ref[1] = jnp.concatenate([a1s[:, 4:8], z12, h1[:, 128:256]], axis=1)
    ad_ref[0] = jnp.concatenate([a1d[:, 0:4], z12], axis=1)
    ad_ref[1] = jnp.concatenate([a1d[:, 4:8], z12], axis=1)
    bs = jnp.max(a1s, axis=0, keepdims=True)
    bd = jnp.max(a1d, axis=0, keepdims=True)

    @pl.when(i == 0)
    def _():
        mxs_ref[...] = bs
        mxd_ref[...] = bd

    @pl.when(i > 0)
    def _():
        mxs_ref[...] = jnp.maximum(mxs_ref[...], bs)
        mxd_ref[...] = jnp.maximum(mxd_ref[...], bd)


def _dense1(x, w1, b1, wc1, s1, d1):
    return pl.pallas_call(
        _dense1_body,
        grid=(N // _BLK,),
        in_specs=[
            pl.BlockSpec((_BLK, IN_CH), lambda i: (i, 0)),
            pl.BlockSpec((IN_CH, HID), lambda i: (0, 0)),
            pl.BlockSpec((HID,), lambda i: (0,)),
            pl.BlockSpec((HID, HEADS * HID), lambda i: (0, 0)),
            pl.BlockSpec((HEADS * HID, HEADS), lambda i: (0, 0)),
            pl.BlockSpec((HEADS * HID, HEADS), lambda i: (0, 0)),
        ],
        out_specs=[
            pl.BlockSpec((2, _BLK, _G1W), lambda i: (0, i, 0)),
            pl.BlockSpec((2, _BLK, 16), lambda i: (0, i, 0)),
            pl.BlockSpec((1, HEADS), lambda i: (0, 0)),
            pl.BlockSpec((1, HEADS), lambda i: (0, 0)),
        ],
        out_shape=[
            jax.ShapeDtypeStruct((2, N, _G1W), jnp.float32),
            jax.ShapeDtypeStruct((2, N, 16), jnp.float32),
            jax.ShapeDtypeStruct((1, HEADS), jnp.float32),
            jax.ShapeDtypeStruct((1, HEADS), jnp.float32),
        ],
    )(x, w1, b1, wc1, s1, d1)


def _sc1_body(edge_ref, g_ref, adt_ref, cvec_ref, out_ref,
              acc, cbuf,
              rb0, rb1, ab0, ab1, sb0, sb1,
              eidx, si0, si1, di0, di1, dS0, dS1, dSt,
              sg0, sg1, sa0, sa1, ss0, ss1):
    cid = lax.axis_index("c")
    sid = lax.axis_index("s")
    coff = cid * N
    pltpu.sync_copy(cvec_ref, cbuf)
    cv = cbuf[cid]

    for j in range((_NCHUNK + 15) // 16):
        cidx = sid + 16 * j

        @pl.when(cidx < _NCHUNK)
        def _():
            rb = cidx * _NB
            pltpu.sync_copy(g_ref.at[pl.ds(coff + rb, _NB)],
                            rb0.at[pl.ds(0, _NB)])
            pltpu.sync_copy(adt_ref.at[pl.ds(coff + rb, _NB)],
                            ab0.at[pl.ds(0, _NB)])

            def _init_row(r, _):
                asv = rb0[r, pl.ds(0, 16)]
                adv = ab0[r, pl.ds(0, 16)]
                z = asv + adv
                al = jnp.where(z > 0, z, 0.2 * z) - cv
                exv = jnp.exp(al)
                rb0[r, pl.ds(0, 16)] = exv
                for h in range(4):
                    exh = exv[h]
                    for k in range(2):
                        c0 = 16 + 32 * h + 16 * k
                        rb0[r, pl.ds(c0, 16)] = exh * rb0[r, pl.ds(c0, 16)]
                return 0

            lax.fori_loop(0, _NB, _init_row, 0)
            pltpu.sync_copy(rb0.at[pl.ds(0, _NB)], acc.at[pl.ds(rb, _NB)])

    plsc.subcore_barrier()

    bufs = ((rb0, ab0, sb0, si0, di0, dS0, sg0, sa0, ss0),
            (rb1, ab1, sb1, si1, di1, dS1, sg1, sa1, ss1))
    tbase = sid * _ET1

    def _refill(q):
        pltpu.sync_copy(edge_ref.at[:, pl.ds(tbase + q * _IFILL, _IFILL)],
                        eidx)

    def _issue(i, bid):
        rbE, abE, sbuf, sidx, didx2, didxS, sg, sa, ss = bufs[i]
        m = 64 * lax.rem(bid, _IFILL // _EB)
        for k in range(_EB // 16):
            sl = pl.ds(16 * k, 16)
            sidx[sl] = eidx[0, pl.ds(m + 16 * k, 16)] + coff
            didx2[sl] = eidx[1, pl.ds(m + 16 * k, 16)] + coff
        pltpu.async_copy(g_ref.at[sidx], rbE, sg)
        pltpu.async_copy(adt_ref.at[didx2], abE, sa)

    def _wait_gather(i):
        rbE, abE, sbuf, sidx, didx2, didxS, sg, sa, ss = bufs[i]
        pltpu.make_async_copy(g_ref.at[sidx], rbE, sg).wait()
        pltpu.make_async_copy(adt_ref.at[didx2], abE, sa).wait()

    def _wait_scatter(i):
        rbE, abE, sbuf, sidx, didx2, didxS, sg, sa, ss = bufs[i]
        pltpu.make_async_copy(sbuf, acc.at[didxS], ss).wait()

    def _compute_scatter(i):
        rbE, abE, sbuf, sidx, didx2, didxS, sg, sa, ss = bufs[i]
        for k in range(_EB // 16):
            sl = pl.ds(16 * k, 16)
            didxS[sl] = didx2[sl] - coff

        def _edge(e, _):
            asv = rbE[e, pl.ds(0, 16)]
            adv = abE[e, pl.ds(0, 16)]
            z = asv + adv
            al = jnp.where(z > 0, z, 0.2 * z) - cv
            exv = jnp.exp(al)
            sbuf[e, pl.ds(0, 16)] = exv
            for h in range(4):
                exh = exv[h]
                for k in range(2):
                    c0 = 16 + 32 * h + 16 * k
                    sbuf[e, pl.ds(c0, 16)] = exh * rbE[e, pl.ds(c0, 16)]
            return 0

        lax.fori_loop(0, _EB, _edge, 0)
        pltpu.async_copy(sbuf, acc.at[didxS], ss, add=True)

    _refill(0)
    _issue(0, 0)
    _issue(1, 1)

    def _pair(p, _):
        @pl.when(lax.rem(p + 1, (_IFILL // _EB) // 2) == 0)
        def _():
            _refill((p + 1) // ((_IFILL // _EB) // 2))

        _wait_gather(0)

        @pl.when(p > 0)
        def _():
            _wait_scatter(0)

        _compute_scatter(0)

        @pl.when(2 * p + 2 < _NF1)
        def _():
            _issue(0, 2 * p + 2)

        _wait_gather(1)

        @pl.when(p > 0)
        def _():
            _wait_scatter(1)

        _compute_scatter(1)

        @pl.when(2 * p + 3 < _NF1)
        def _():
            _issue(1, 2 * p + 3)

        return 0

    lax.fori_loop(0, _P1, _pair, 0)
    _wait_scatter(0)
    _wait_scatter(1)

    for k in range(_TL1 // 16):
        sl = pl.ds(16 * k, 16)
        si0[sl] = eidx[0, pl.ds(16 * k, 16)] + coff
        di0[sl] = eidx[1, pl.ds(16 * k, 16)] + coff
        dSt[sl] = eidx[1, pl.ds(16 * k, 16)]
    pltpu.async_copy(g_ref.at[si0.at[pl.ds(0, _TL1)]],
                     rb0.at[pl.ds(0, _TL1)], sg0)
    pltpu.async_copy(adt_ref.at[di0.at[pl.ds(0, _TL1)]],
                     ab0.at[pl.ds(0, _TL1)], sa0)
    pltpu.make_async_copy(g_ref.at[si0.at[pl.ds(0, _TL1)]],
                          rb0.at[pl.ds(0, _TL1)], sg0).wait()
    pltpu.make_async_copy(adt_ref.at[di0.at[pl.ds(0, _TL1)]],
                          ab0.at[pl.ds(0, _TL1)], sa0).wait()

    def _tail_edge(e, _):
        asv = rb0[e, pl.ds(0, 16)]
        adv = ab0[e, pl.ds(0, 16)]
        z = asv + adv
        al = jnp.where(z > 0, z, 0.2 * z) - cv
        exv = jnp.exp(al)
        sb0[e, pl.ds(0, 16)] = exv
        for h in range(4):
            exh = exv[h]
            for k in range(2):
                c0 = 16 + 32 * h + 16 * k
                sb0[e, pl.ds(c0, 16)] = exh * rb0[e, pl.ds(c0, 16)]
        return 0

    lax.fori_loop(0, _TL1, _tail_edge, 0)
    pltpu.async_copy(sb0.at[pl.ds(0, _TL1)],
                     acc.at[dSt], ss0, add=True)
    pltpu.make_async_copy(sb0.at[pl.ds(0, _TL1)],
                          acc.at[dSt], ss0).wait()
    plsc.subcore_barrier()

    for j in range((_NCHUNK + 15) // 16):
        cidx = sid + 16 * j

        @pl.when(cidx < _NCHUNK)
        def _():
            rb = cidx * _NB
            pltpu.sync_copy(acc.at[pl.ds(rb, _NB)], rb0.at[pl.ds(0, _NB)])
            pltpu.sync_copy(rb0.at[pl.ds(0, _NB)],
                            out_ref.at[cid, pl.ds(rb, _NB)])


def _sc1(edge_index, g, adt, cvec):
    mesh = plsc.VectorSubcoreMesh(core_axis_name="c", subcore_axis_name="s")
    f = pl.kernel(
        _sc1_body,
        out_type=jax.ShapeDtypeStruct((2, N, _G1W), jnp.float32),
        mesh=mesh,
        scratch_types=[
            pltpu.VMEM_SHARED((N, _G1W), jnp.float32),
            pltpu.VMEM((2, 16), jnp.float32),
            pltpu.VMEM((_EB, _G1W), jnp.float32),
            pltpu.VMEM((_EB, _G1W), jnp.float32),
            pltpu.VMEM((_EB, 16), jnp.float32),
            pltpu.VMEM((_EB, 16), jnp.float32),
            pltpu.VMEM((_EB, _G1W), jnp.float32),
            pltpu.VMEM((_EB, _G1W), jnp.float32),
            pltpu.VMEM((2, _IFILL), jnp.int32),
            pltpu.VMEM((_EB,), jnp.int32),
            pltpu.VMEM((_EB,), jnp.int32),
            pltpu.VMEM((_EB,), jnp.int32),
            pltpu.VMEM((_EB,), jnp.int32),
            pltpu.VMEM((_EB,), jnp.int32),
            pltpu.VMEM((_EB,), jnp.int32),
            pltpu.VMEM((_TL1,), jnp.int32),
            pltpu.SemaphoreType.DMA,
            pltpu.SemaphoreType.DMA,
            pltpu.SemaphoreType.DMA,
            pltpu.SemaphoreType.DMA,
            pltpu.SemaphoreType.DMA,
            pltpu.SemaphoreType.DMA,
        ],
        compiler_params=pltpu.CompilerParams(use_tc_tiling_on_sc=False),
    )
    return f(edge_index, g, adt, cvec)


def _dense2_body(acc_ref, bc1_ref, wc2_ref, s2_ref, d2_ref,
                 t2_ref, ad2_ref, mxs_ref, mxd_ref):
    i = pl.program_id(0)
    a0 = acc_ref[0]
    a1 = acc_ref[1]
    msg = jnp.concatenate([a0[:, 16:_G1W], a1[:, 16:_G1W]], axis=1)
    dens = []
    for c in range(2):
        a = a0 if c == 0 else a1
        for h in range(4):
            dens.append(jnp.broadcast_to(a[:, h:h + 1], (_BLK, HID)))
    den = jnp.concatenate(dens, axis=1)
    g1 = msg / (den + 1e-16) + bc1_ref[...][None, :]
    g1 = jnp.where(g1 > 0, g1, jnp.exp(g1) - 1.0)
    h2 = jnp.dot(g1, wc2_ref[...], preferred_element_type=jnp.float32)
    a2s = jnp.dot(h2, s2_ref[...], preferred_element_type=jnp.float32)
    a2d = jnp.dot(h2, d2_ref[...], preferred_element_type=jnp.float32)
    z15 = jnp.zeros((_BLK, 15), jnp.float32)
    t2_ref[...] = jnp.concatenate([a2s[:, 0:1], z15, h2], axis=1)
    ad2_ref[...] = jnp.concatenate([a2d[:, 0:1], z15], axis=1)
    bs = jnp.max(a2s, axis=0, keepdims=True)
    bd = jnp.max(a2d, axis=0, keepdims=True)

    @pl.when(i == 0)
    def _():
        mxs_ref[...] = bs
        mxd_ref[...] = bd

    @pl.when(i > 0)
    def _():
        mxs_ref[...] = jnp.maximum(mxs_ref[...], bs)
        mxd_ref[...] = jnp.maximum(mxd_ref[...], bd)


def _dense2(acc1, bc1, wc2, s2, d2):
    return pl.pallas_call(
        _dense2_body,
        grid=(N // _BLK,),
        in_specs=[
            pl.BlockSpec((2, _BLK, _G1W), lambda i: (0, i, 0)),
            pl.BlockSpec((HEADS * HID,), lambda i: (0,)),
            pl.BlockSpec((HEADS * HID, HID), lambda i: (0, 0)),
            pl.BlockSpec((HID, 8), lambda i: (0, 0)),
            pl.BlockSpec((HID, 8), lambda i: (0, 0)),
        ],
        out_specs=[
            pl.BlockSpec((_BLK, _G2W), lambda i: (i, 0)),
            pl.BlockSpec((_BLK, 16), lambda i: (i, 0)),
            pl.BlockSpec((1, 8), lambda i: (0, 0)),
            pl.BlockSpec((1, 8), lambda i: (0, 0)),
        ],
        out_shape=[
            jax.ShapeDtypeStruct((N, _G2W), jnp.float32),
            jax.ShapeDtypeStruct((N, 16), jnp.float32),
            jax.ShapeDtypeStruct((1, 8), jnp.float32),
            jax.ShapeDtypeStruct((1, 8), jnp.float32),
        ],
    )(acc1, bc1, wc2, s2, d2)


def _sc2_body(edge_ref, t2_ref, ad2_ref, cvec_ref, out_ref,
              acc, cbuf,
              rb0, rb1, ab0, ab1, sb0, sb1,
              eidx, si0, si1, di0, di1, dS0, dS1, dSt,
              sg0, sg1, sa0, sa1, ss0, ss1):
    cid = lax.axis_index("c")
    sid = lax.axis_index("s")
    pltpu.sync_copy(cvec_ref, cbuf)
    cv = cbuf[...]
    scale = jnp.where(cid == 0, 1.0, 0.0)

    for j in range((_NCHUNK + 15) // 16):
        cidx = sid + 16 * j

        @pl.when(cidx < _NCHUNK)
        def _():
            rb = cidx * _NB
            pltpu.sync_copy(t2_ref.at[pl.ds(rb, _NB)], rb0.at[pl.ds(0, _NB)])
            pltpu.sync_copy(ad2_ref.at[pl.ds(rb, _NB)], ab0.at[pl.ds(0, _NB)])

            def _init_row(r, _):
                asv = rb0[r, pl.ds(0, 16)]
                adv = ab0[r, pl.ds(0, 16)]
                z = asv + adv
                al = jnp.where(z > 0, z, 0.2 * z) - cv
                exv = jnp.exp(al) * scale
                rb0[r, pl.ds(0, 16)] = exv
                ex0 = exv[0]
                for k in range(2):
                    c0 = 16 + 16 * k
                    rb0[r, pl.ds(c0, 16)] = ex0 * rb0[r, pl.ds(c0, 16)]
                return 0

            lax.fori_loop(0, _NB, _init_row, 0)
            pltpu.sync_copy(rb0.at[pl.ds(0, _NB)], acc.at[pl.ds(rb, _NB)])

    plsc.subcore_barrier()

    bufs = ((rb0, ab0, sb0, si0, di0, dS0, sg0, sa0, ss0),
            (rb1, ab1, sb1, si1, di1, dS1, sg1, sa1, ss1))
    tbase = cid * (E // 2) + sid * _ET2

    def _refill(q):
        pltpu.sync_copy(edge_ref.at[:, pl.ds(tbase + q * _IFILL, _IFILL)],
                        eidx)

    def _issue(i, bid):
        rbE, abE, sbuf, sidx, didx2, didxS, sg, sa, ss = bufs[i]
        m = _EB2 * lax.rem(bid, _IFILL // _EB2)
        for k in range(_EB2 // 16):
            sl = pl.ds(16 * k, 16)
            sidx[sl] = eidx[0, pl.ds(m + 16 * k, 16)]
            didx2[sl] = eidx[1, pl.ds(m + 16 * k, 16)]
        pltpu.async_copy(t2_ref.at[sidx], rbE, sg)
        pltpu.async_copy(ad2_ref.at[didx2], abE, sa)

    def _wait_gather(i):
        rbE, abE, sbuf, sidx, didx2, didxS, sg, sa, ss = bufs[i]
        pltpu.make_async_copy(t2_ref.at[sidx], rbE, sg).wait()
        pltpu.make_async_copy(ad2_ref.at[didx2], abE, sa).wait()

    def _wait_scatter(i):
        rbE, abE, sbuf, sidx, didx2, didxS, sg, sa, ss = bufs[i]
        pltpu.make_async_copy(sbuf, acc.at[didxS], ss).wait()

    def _compute_scatter(i):
        rbE, abE, sbuf, sidx, didx2, didxS, sg, sa, ss = bufs[i]
        for k in range(_EB2 // 16):
            sl = pl.ds(16 * k, 16)
            didxS[sl] = didx2[sl]

        def _edge(e, _):
            asv = rbE[e, pl.ds(0, 16)]
            adv = abE[e, pl.ds(0, 16)]
            z = asv + adv
            al = jnp.where(z > 0, z, 0.2 * z) - cv
            exv = jnp.exp(al)
            sbuf[e, pl.ds(0, 16)] = exv
            ex0 = exv[0]
            for k in range(2):
                c0 = 16 + 16 * k
                sbuf[e, pl.ds(c0, 16)] = ex0 * rbE[e, pl.ds(c0, 16)]
            return 0

        lax.fori_loop(0, _EB2, _edge, 0)
        pltpu.async_copy(sbuf, acc.at[didxS], ss, add=True)

    _refill(0)
    _issue(0, 0)
    _issue(1, 1)

    def _pair(p, _):
        @pl.when(lax.rem(p + 1, (_IFILL // _EB2) // 2) == 0)
        def _():
            _refill((p + 1) // ((_IFILL // _EB2) // 2))

        _wait_gather(0)

        @pl.when(p > 0)
        def _():
            _wait_scatter(0)

        _compute_scatter(0)

        @pl.when(2 * p + 2 < _NF2)
        def _():
            _issue(0, 2 * p + 2)

        _wait_gather(1)

        @pl.when(p > 0)
        def _():
            _wait_scatter(1)

        _compute_scatter(1)

        @pl.when(2 * p + 3 < _NF2)
        def _():
            _issue(1, 2 * p + 3)

        return 0

    lax.fori_loop(0, _P2, _pair, 0)
    _wait_scatter(0)
    _wait_scatter(1)

    for k in range(_TL2 // 16):
        sl = pl.ds(16 * k, 16)
        si0[sl] = eidx[0, pl.ds(_TC2 + 16 * k, 16)]
        dSt[sl] = eidx[1, pl.ds(_TC2 + 16 * k, 16)]
    pltpu.async_copy(t2_ref.at[si0.at[pl.ds(0, _TL2)]],
                     rb0.at[pl.ds(0, _TL2)], sg0)
    pltpu.async_copy(ad2_ref.at[dSt], ab0.at[pl.ds(0, _TL2)], sa0)
    pltpu.make_async_copy(t2_ref.at[si0.at[pl.ds(0, _TL2)]],
                          rb0.at[pl.ds(0, _TL2)], sg0).wait()
    pltpu.make_async_copy(ad2_ref.at[dSt], ab0.at[pl.ds(0, _TL2)],
                          sa0).wait()

    def _tail_edge(e, _):
        asv = rb0[e, pl.ds(0, 16)]
        adv = ab0[e, pl.ds(0, 16)]
        z = asv + adv
        al = jnp.where(z > 0, z, 0.2 * z) - cv
        exv = jnp.exp(al)
        sb0[e, pl.ds(0, 16)] = exv
        ex0 = exv[0]
        for k in range(2):
            c0 = 16 + 16 * k
            sb0[e, pl.ds(c0, 16)] = ex0 * rb0[e, pl.ds(c0, 16)]
        return 0

    lax.fori_loop(0, _TL2, _tail_edge, 0)
    pltpu.async_copy(sb0.at[pl.ds(0, _TL2)], acc.at[dSt], ss0, add=True)
    pltpu.make_async_copy(sb0.at[pl.ds(0, _TL2)], acc.at[dSt], ss0).wait()
    plsc.subcore_barrier()

    for j in range((_NCHUNK + 15) // 16):
        cidx = sid + 16 * j

        @pl.when(cidx < _NCHUNK)
        def _():
            rb = cidx * _NB
            pltpu.sync_copy(acc.at[pl.ds(rb, _NB)], rb0.at[pl.ds(0, _NB)])
            pltpu.sync_copy(rb0.at[pl.ds(0, _NB)],
                            out_ref.at[cid, pl.ds(rb, _NB)])


def _sc2(edge_index, t2, ad2, cvec):
    mesh = plsc.VectorSubcoreMesh(core_axis_name="c", subcore_axis_name="s")
    f = pl.kernel(
        _sc2_body,
        out_type=jax.ShapeDtypeStruct((2, N, _G2W), jnp.float32),
        mesh=mesh,
        scratch_types=[
            pltpu.VMEM_SHARED((N, _G2W), jnp.float32),
            pltpu.VMEM((16,), jnp.float32),
            pltpu.VMEM((_EB2, _G2W), jnp.float32),
            pltpu.VMEM((_EB2, _G2W), jnp.float32),
            pltpu.VMEM((_EB2, 16), jnp.float32),
            pltpu.VMEM((_EB2, 16), jnp.float32),
            pltpu.VMEM((_EB2, _G2W), jnp.float32),
            pltpu.VMEM((_EB2, _G2W), jnp.float32),
            pltpu.VMEM((2, _IFILL), jnp.int32),
            pltpu.VMEM((_EB2,), jnp.int32),
            pltpu.VMEM((_EB2,), jnp.int32),
            pltpu.VMEM((_EB2,), jnp.int32),
            pltpu.VMEM((_EB2,), jnp.int32),
            pltpu.VMEM((_EB2,), jnp.int32),
            pltpu.VMEM((_EB2,), jnp.int32),
            pltpu.VMEM((_TL2,), jnp.int32),
            pltpu.SemaphoreType.DMA,
            pltpu.SemaphoreType.DMA,
            pltpu.SemaphoreType.DMA,
            pltpu.SemaphoreType.DMA,
            pltpu.SemaphoreType.DMA,
            pltpu.SemaphoreType.DMA,
        ],
        compiler_params=pltpu.CompilerParams(use_tc_tiling_on_sc=False),
    )
    return f(edge_index, t2, ad2, cvec)


def _dense3_body(acc_ref, bc2_ref, w2_ref, b2_ref, out_ref):
    a0 = acc_ref[0]
    a1 = acc_ref[1]
    msg = a0[:, 16:_G2W] + a1[:, 16:_G2W]
    den = jnp.broadcast_to(a0[:, 0:1] + a1[:, 0:1], (_BLK, HID))
    g2 = msg / (den + 1e-16) + bc2_ref[...][None, :]
    out_ref[...] = (
        jnp.dot(g2, w2_ref[...], preferred_element_type=jnp.float32)
        + b2_ref[...][None, :])


def _dense3(acc2, bc2, w2, b2):
    return pl.pallas_call(
        _dense3_body,
        grid=(N // _BLK,),
        in_specs=[
            pl.BlockSpec((2, _BLK, _G2W), lambda i: (0, i, 0)),
            pl.BlockSpec((HID,), lambda i: (0,)),
            pl.BlockSpec((HID, OUT_CH), lambda i: (0, 0)),
            pl.BlockSpec((OUT_CH,), lambda i: (0,)),
        ],
        out_specs=pl.BlockSpec((_BLK, OUT_CH), lambda i: (i, 0)),
        out_shape=jax.ShapeDtypeStruct((N, OUT_CH), jnp.float32),
    )(acc2, bc2, w2, b2)


def kernel(x, edge_index, W_lin1, b_lin1, Wc1, bc1, attc1_s, attc1_d,
           Wc2, bc2, attc2_s, attc2_d, W_lin2, b_lin2):
    eye = jnp.eye(HEADS, dtype=jnp.float32)
    s1 = (eye[:, None, :] * attc1_s.T[None, :, :]).reshape(HEADS * HID, HEADS)
    d1 = (eye[:, None, :] * attc1_d.T[None, :, :]).reshape(HEADS * HID, HEADS)
    s2 = jnp.pad(attc2_s.T, ((0, 0), (0, 7)))
    d2 = jnp.pad(attc2_d.T, ((0, 0), (0, 7)))

    epad = jnp.pad(edge_index, ((0, 0), (0, _EPAD - E)))

    g, adt, mxs, mxd = _dense1(x, W_lin1, b_lin1, Wc1, s1, d1)
    c1 = jnp.maximum(mxs[0] + mxd[0], 0.0)
    cvec1 = jnp.zeros((2, 16), jnp.float32).at[:, :4].set(c1.reshape(2, 4))
    acc1 = _sc1(epad, g.reshape(2 * N, _G1W),
                adt.reshape(2 * N, 16), cvec1)

    t2, ad2, mxs2, mxd2 = _dense2(acc1, bc1, Wc2, s2, d2)
    c2 = jnp.maximum(mxs2[0, 0] + mxd2[0, 0], 0.0)
    cvec2 = jnp.zeros((16,), jnp.float32).at[0].set(c2)
    acc2 = _sc2(epad, t2, ad2, cvec2)

    return _dense3(acc2, bc2, W_lin2, b_lin2)


# speedup vs baseline: 2.6586x; 1.5716x over previous
import functools

import jax
import jax.numpy as jnp
from jax import lax
from jax.experimental import pallas as pl
from jax.experimental.pallas import tpu as pltpu
from jax.experimental.pallas import tpu_sc as plsc

N = 10000
E = 320000
IN_CH = 128
HID = 32
OUT_CH = 128
HEADS = 8

_BLK = 1000
_EB = 64
_NB = 40
_NCHUNK = N // _NB

_IFILL = 512
_EPAD = E + _IFILL

_ET1 = E // 16
_NF1 = _ET1 // _EB
_P1 = _NF1 // 2
_TL1 = _ET1 - _NF1 * _EB

_EB2 = 128
_ET2 = E // 32
_NF2 = _ET2 // _EB2
_P2 = _NF2 // 2
_TL2 = _ET2 - _NF2 * _EB2
_TC2 = (_NF2 * _EB2) % _IFILL

_G1W = 144
_G2W = 48


def _dense1_body(x_ref, w1_ref, b1_ref, wc1_ref, s1_ref, d1_ref,
                 g_ref, ad_ref, mxs_ref, mxd_ref):
    i = pl.program_id(0)
    h0 = jax.nn.relu(
        jnp.dot(x_ref[...], w1_ref[...], preferred_element_type=jnp.float32)
        + b1_ref[...][None, :])
    h1 = jnp.dot(h0, wc1_ref[...], preferred_element_type=jnp.float32)
    a1s = jnp.dot(h1, s1_ref[...], preferred_element_type=jnp.float32)
    a1d = jnp.dot(h1, d1_ref[...], preferred_element_type=jnp.float32)
    z12 = jnp.zeros((_BLK, 12), jnp.float32)
    g_ref[0] = jnp.concatenate([a1s[:, 0:4], z12, h1[:, 0:128]], axis=1)
    g_ref[1] = jnp.concatenate([a1s[:, 4:8], z12, h1[:, 128:256]], axis=1)
    ad_ref[0] = jnp.concatenate([a1d[:, 0:4], z12], axis=1)
    ad_ref[1] = jnp.concatenate([a1d[:, 4:8], z12], axis=1)
    bs = jnp.max(a1s, axis=0, keepdims=True)
    bd = jnp.max(a1d, axis=0, keepdims=True)

    @pl.when(i == 0)
    def _():
        mxs_ref[...] = bs
        mxd_ref[...] = bd

    @pl.when(i > 0)
    def _():
        mxs_ref[...] = jnp.maximum(mxs_ref[...], bs)
        mxd_ref[...] = jnp.maximum(mxd_ref[...], bd)


def _dense1(x, w1, b1, wc1, s1, d1):
    return pl.pallas_call(
        _dense1_body,
        grid=(N // _BLK,),
        in_specs=[
            pl.BlockSpec((_BLK, IN_CH), lambda i: (i, 0)),
            pl.BlockSpec((IN_CH, HID), lambda i: (0, 0)),
            pl.BlockSpec((HID,), lambda i: (0,)),
            pl.BlockSpec((HID, HEADS * HID), lambda i: (0, 0)),
            pl.BlockSpec((HEADS * HID, HEADS), lambda i: (0, 0)),
            pl.BlockSpec((HEADS * HID, HEADS), lambda i: (0, 0)),
        ],
        out_specs=[
            pl.BlockSpec((2, _BLK, _G1W), lambda i: (0, i, 0)),
            pl.BlockSpec((2, _BLK, 16), lambda i: (0, i, 0)),
            pl.BlockSpec((1, HEADS), lambda i: (0, 0)),
            pl.BlockSpec((1, HEADS), lambda i: (0, 0)),
        ],
        out_shape=[
            jax.ShapeDtypeStruct((2, N, _G1W), jnp.float32),
            jax.ShapeDtypeStruct((2, N, 16), jnp.float32),
            jax.ShapeDtypeStruct((1, HEADS), jnp.float32),
            jax.ShapeDtypeStruct((1, HEADS), jnp.float32),
        ],
    )(x, w1, b1, wc1, s1, d1)


def _sc1_body(edge_ref, g_ref, adt_ref, cvec_ref, out_ref,
              acc, cbuf,
              rb0, rb1, ab0, ab1, sb0, sb1,
              eidx, si0, si1, di0, di1, dS0, dS1, dSt,
              sg0, sg1, sa0, sa1, ss0, ss1):
    cid = lax.axis_index("c")
    sid = lax.axis_index("s")
    coff = cid * N
    pltpu.sync_copy(cvec_ref, cbuf)
    cv = cbuf[cid]

    for j in range((_NCHUNK + 15) // 16):
        cidx = sid + 16 * j

        @pl.when(cidx < _NCHUNK)
        def _():
            rb = cidx * _NB
            pltpu.sync_copy(g_ref.at[pl.ds(coff + rb, _NB)],
                            rb0.at[pl.ds(0, _NB)])
            pltpu.sync_copy(adt_ref.at[pl.ds(coff + rb, _NB)],
                            ab0.at[pl.ds(0, _NB)])

            def _init_row(r, _):
                asv = rb0[r, pl.ds(0, 16)]
                adv = ab0[r, pl.ds(0, 16)]
                z = asv + adv
                al = jnp.where(z > 0, z, 0.2 * z) - cv
                exv = jnp.exp(al)
                rb0[r, pl.ds(0, 16)] = exv
                for h in range(4):
                    exh = exv[h]
                    for k in range(2):
                        c0 = 16 + 32 * h + 16 * k
                        rb0[r, pl.ds(c0, 16)] = exh * rb0[r, pl.ds(c0, 16)]
                return 0

            lax.fori_loop(0, _NB, _init_row, 0)
            pltpu.sync_copy(rb0.at[pl.ds(0, _NB)], acc.at[pl.ds(rb, _NB)])

    plsc.subcore_barrier()

    bufs = ((rb0, ab0, sb0, si0, di0, dS0, sg0, sa0, ss0),
            (rb1, ab1, sb1, si1, di1, dS1, sg1, sa1, ss1))
    tbase = sid * _ET1

    def _refill(q):
        pltpu.sync_copy(edge_ref.at[:, pl.ds(tbase + q * _IFILL, _IFILL)],
                        eidx)

    def _issue(i, bid):
        rbE, abE, sbuf, sidx, didx2, didxS, sg, sa, ss = bufs[i]
        m = 64 * lax.rem(bid, _IFILL // _EB)
        for k in range(_EB // 16):
            sl = pl.ds(16 * k, 16)
            sidx[sl] = eidx[0, pl.ds(m + 16 * k, 16)] + coff
            didx2[sl] = eidx[1, pl.ds(m + 16 * k, 16)] + coff
        pltpu.async_copy(g_ref.at[sidx], rbE, sg)
        pltpu.async_copy(adt_ref.at[didx2], abE, sa)

    def _wait_gather(i):
        rbE, abE, sbuf, sidx, didx2, didxS, sg, sa, ss = bufs[i]
        pltpu.make_async_copy(g_ref.at[sidx], rbE, sg).wait()
        pltpu.make_async_copy(adt_ref.at[didx2], abE, sa).wait()

    def _wait_scatter(i):
        rbE, abE, sbuf, sidx, didx2, didxS, sg, sa, ss = bufs[i]
        pltpu.make_async_copy(sbuf, acc.at[didxS], ss).wait()

    def _compute_scatter(i):
        rbE, abE, sbuf, sidx, didx2, didxS, sg, sa, ss = bufs[i]
        for k in range(_EB // 16):
            sl = pl.ds(16 * k, 16)
            didxS[sl] = didx2[sl] - coff

        @plsc.parallel_loop(0, _EB, unroll=4)
        def _edge(e):
            asv = rbE[e, pl.ds(0, 16)]
            adv = abE[e, pl.ds(0, 16)]
            z = asv + adv
            al = jnp.where(z > 0, z, 0.2 * z) - cv
            exv = jnp.exp(al)
            sbuf[e, pl.ds(0, 16)] = exv
            for h in range(4):
                exh = exv[h]
                for k in range(2):
                    c0 = 16 + 32 * h + 16 * k
                    sbuf[e, pl.ds(c0, 16)] = exh * rbE[e, pl.ds(c0, 16)]

        pltpu.async_copy(sbuf, acc.at[didxS], ss, add=True)

    _refill(0)
    _issue(0, 0)
    _issue(1, 1)

    def _pair(p, _):
        @pl.when(lax.rem(p + 1, (_IFILL // _EB) // 2) == 0)
        def _():
            _refill((p + 1) // ((_IFILL // _EB) // 2))

        _wait_gather(0)

        @pl.when(p > 0)
        def _():
            _wait_scatter(0)

        _compute_scatter(0)

        @pl.when(2 * p + 2 < _NF1)
        def _():
            _issue(0, 2 * p + 2)

        _wait_gather(1)

        @pl.when(p > 0)
        def _():
            _wait_scatter(1)

        _compute_scatter(1)

        @pl.when(2 * p + 3 < _NF1)
        def _():
            _issue(1, 2 * p + 3)

        return 0

    lax.fori_loop(0, _P1, _pair, 0)
    _wait_scatter(0)
    _wait_scatter(1)

    for k in range(_TL1 // 16):
        sl = pl.ds(16 * k, 16)
        si0[sl] = eidx[0, pl.ds(16 * k, 16)] + coff
        di0[sl] = eidx[1, pl.ds(16 * k, 16)] + coff
        dSt[sl] = eidx[1, pl.ds(16 * k, 16)]
    pltpu.async_copy(g_ref.at[si0.at[pl.ds(0, _TL1)]],
                     rb0.at[pl.ds(0, _TL1)], sg0)
    pltpu.async_copy(adt_ref.at[di0.at[pl.ds(0, _TL1)]],
                     ab0.at[pl.ds(0, _TL1)], sa0)
    pltpu.make_async_copy(g_ref.at[si0.at[pl.ds(0, _TL1)]],
                          rb0.at[pl.ds(0, _TL1)], sg0).wait()
    pltpu.make_async_copy(adt_ref.at[di0.at[pl.ds(0, _TL1)]],
                          ab0.at[pl.ds(0, _TL1)], sa0).wait()

    def _tail_edge(e, _):
        asv = rb0[e, pl.ds(0, 16)]
        adv = ab0[e, pl.ds(0, 16)]
        z = asv + adv
        al = jnp.where(z > 0, z, 0.2 * z) - cv
        exv = jnp.exp(al)
        sb0[e, pl.ds(0, 16)] = exv
        for h in range(4):
            exh = exv[h]
            for k in range(2):
                c0 = 16 + 32 * h + 16 * k
                sb0[e, pl.ds(c0, 16)] = exh * rb0[e, pl.ds(c0, 16)]
        return 0

    lax.fori_loop(0, _TL1, _tail_edge, 0)
    pltpu.async_copy(sb0.at[pl.ds(0, _TL1)],
                     acc.at[dSt], ss0, add=True)
    pltpu.make_async_copy(sb0.at[pl.ds(0, _TL1)],
                          acc.at[dSt], ss0).wait()
    plsc.subcore_barrier()

    for j in range((_NCHUNK + 15) // 16):
        cidx = sid + 16 * j

        @pl.when(cidx < _NCHUNK)
        def _():
            rb = cidx * _NB
            pltpu.sync_copy(acc.at[pl.ds(rb, _NB)], rb0.at[pl.ds(0, _NB)])
            pltpu.sync_copy(rb0.at[pl.ds(0, _NB)],
                            out_ref.at[cid, pl.ds(rb, _NB)])


def _sc1(edge_index, g, adt, cvec):
    mesh = plsc.VectorSubcoreMesh(core_axis_name="c", subcore_axis_name="s")
    f = pl.kernel(
        _sc1_body,
        out_type=jax.ShapeDtypeStruct((2, N, _G1W), jnp.float32),
        mesh=mesh,
        scratch_types=[
            pltpu.VMEM_SHARED((N, _G1W), jnp.float32),
            pltpu.VMEM((2, 16), jnp.float32),
            pltpu.VMEM((_EB, _G1W), jnp.float32),
            pltpu.VMEM((_EB, _G1W), jnp.float32),
            pltpu.VMEM((_EB, 16), jnp.float32),
            pltpu.VMEM((_EB, 16), jnp.float32),
            pltpu.VMEM((_EB, _G1W), jnp.float32),
            pltpu.VMEM((_EB, _G1W), jnp.float32),
            pltpu.VMEM((2, _IFILL), jnp.int32),
            pltpu.VMEM((_EB,), jnp.int32),
            pltpu.VMEM((_EB,), jnp.int32),
            pltpu.VMEM((_EB,), jnp.int32),
            pltpu.VMEM((_EB,), jnp.int32),
            pltpu.VMEM((_EB,), jnp.int32),
            pltpu.VMEM((_EB,), jnp.int32),
            pltpu.VMEM((_TL1,), jnp.int32),
            pltpu.SemaphoreType.DMA,
            pltpu.SemaphoreType.DMA,
            pltpu.SemaphoreType.DMA,
            pltpu.SemaphoreType.DMA,
            pltpu.SemaphoreType.DMA,
            pltpu.SemaphoreType.DMA,
        ],
        compiler_params=pltpu.CompilerParams(use_tc_tiling_on_sc=False),
    )
    return f(edge_index, g, adt, cvec)


def _dense2_body(acc_ref, bc1_ref, wc2_ref, s2_ref, d2_ref,
                 t2_ref, ad2_ref, mxs_ref, mxd_ref):
    i = pl.program_id(0)
    a0 = acc_ref[0]
    a1 = acc_ref[1]
    msg = jnp.concatenate([a0[:, 16:_G1W], a1[:, 16:_G1W]], axis=1)
    dens = []
    for c in range(2):
        a = a0 if c == 0 else a1
        for h in range(4):
            dens.append(jnp.broadcast_to(a[:, h:h + 1], (_BLK, HID)))
    den = jnp.concatenate(dens, axis=1)
    g1 = msg / (den + 1e-16) + bc1_ref[...][None, :]
    g1 = jnp.where(g1 > 0, g1, jnp.exp(g1) - 1.0)
    h2 = jnp.dot(g1, wc2_ref[...], preferred_element_type=jnp.float32)
    a2s = jnp.dot(h2, s2_ref[...], preferred_element_type=jnp.float32)
    a2d = jnp.dot(h2, d2_ref[...], preferred_element_type=jnp.float32)
    z15 = jnp.zeros((_BLK, 15), jnp.float32)
    t2_ref[...] = jnp.concatenate([a2s[:, 0:1], z15, h2], axis=1)
    ad2_ref[...] = jnp.concatenate([a2d[:, 0:1], z15], axis=1)
    bs = jnp.max(a2s, axis=0, keepdims=True)
    bd = jnp.max(a2d, axis=0, keepdims=True)

    @pl.when(i == 0)
    def _():
        mxs_ref[...] = bs
        mxd_ref[...] = bd

    @pl.when(i > 0)
    def _():
        mxs_ref[...] = jnp.maximum(mxs_ref[...], bs)
        mxd_ref[...] = jnp.maximum(mxd_ref[...], bd)


def _dense2(acc1, bc1, wc2, s2, d2):
    return pl.pallas_call(
        _dense2_body,
        grid=(N // _BLK,),
        in_specs=[
            pl.BlockSpec((2, _BLK, _G1W), lambda i: (0, i, 0)),
            pl.BlockSpec((HEADS * HID,), lambda i: (0,)),
            pl.BlockSpec((HEADS * HID, HID), lambda i: (0, 0)),
            pl.BlockSpec((HID, 8), lambda i: (0, 0)),
            pl.BlockSpec((HID, 8), lambda i: (0, 0)),
        ],
        out_specs=[
            pl.BlockSpec((_BLK, _G2W), lambda i: (i, 0)),
            pl.BlockSpec((_BLK, 16), lambda i: (i, 0)),
            pl.BlockSpec((1, 8), lambda i: (0, 0)),
            pl.BlockSpec((1, 8), lambda i: (0, 0)),
        ],
        out_shape=[
            jax.ShapeDtypeStruct((N, _G2W), jnp.float32),
            jax.ShapeDtypeStruct((N, 16), jnp.float32),
            jax.ShapeDtypeStruct((1, 8), jnp.float32),
            jax.ShapeDtypeStruct((1, 8), jnp.float32),
        ],
    )(acc1, bc1, wc2, s2, d2)


def _sc2_body(edge_ref, t2_ref, ad2_ref, cvec_ref, out_ref,
              acc, cbuf,
              rb0, rb1, ab0, ab1, sb0, sb1,
              eidx, si0, si1, di0, di1, dS0, dS1, dSt,
              sg0, sg1, sa0, sa1, ss0, ss1):
    cid = lax.axis_index("c")
    sid = lax.axis_index("s")
    pltpu.sync_copy(cvec_ref, cbuf)
    cv = cbuf[...]
    scale = jnp.where(cid == 0, 1.0, 0.0)

    for j in range((_NCHUNK + 15) // 16):
        cidx = sid + 16 * j

        @pl.when(cidx < _NCHUNK)
        def _():
            rb = cidx * _NB
            pltpu.sync_copy(t2_ref.at[pl.ds(rb, _NB)], rb0.at[pl.ds(0, _NB)])
            pltpu.sync_copy(ad2_ref.at[pl.ds(rb, _NB)], ab0.at[pl.ds(0, _NB)])

            def _init_row(r, _):
                asv = rb0[r, pl.ds(0, 16)]
                adv = ab0[r, pl.ds(0, 16)]
                z = asv + adv
                al = jnp.where(z > 0, z, 0.2 * z) - cv
                exv = jnp.exp(al) * scale
                rb0[r, pl.ds(0, 16)] = exv
                ex0 = exv[0]
                for k in range(2):
                    c0 = 16 + 16 * k
                    rb0[r, pl.ds(c0, 16)] = ex0 * rb0[r, pl.ds(c0, 16)]
                return 0

            lax.fori_loop(0, _NB, _init_row, 0)
            pltpu.sync_copy(rb0.at[pl.ds(0, _NB)], acc.at[pl.ds(rb, _NB)])

    plsc.subcore_barrier()

    bufs = ((rb0, ab0, sb0, si0, di0, dS0, sg0, sa0, ss0),
            (rb1, ab1, sb1, si1, di1, dS1, sg1, sa1, ss1))
    tbase = cid * (E // 2) + sid * _ET2

    def _refill(q):
        pltpu.sync_copy(edge_ref.at[:, pl.ds(tbase + q * _IFILL, _IFILL)],
                        eidx)

    def _issue(i, bid):
        rbE, abE, sbuf, sidx, didx2, didxS, sg, sa, ss = bufs[i]
        m = _EB2 * lax.rem(bid, _IFILL // _EB2)
        for k in range(_EB2 // 16):
            sl = pl.ds(16 * k, 16)
            sidx[sl] = eidx[0, pl.ds(m + 16 * k, 16)]
            didx2[sl] = eidx[1, pl.ds(m + 16 * k, 16)]
        pltpu.async_copy(t2_ref.at[sidx], rbE, sg)
        pltpu.async_copy(ad2_ref.at[didx2], abE, sa)

    def _wait_gather(i):
        rbE, abE, sbuf, sidx, didx2, didxS, sg, sa, ss = bufs[i]
        pltpu.make_async_copy(t2_ref.at[sidx], rbE, sg).wait()
        pltpu.make_async_copy(ad2_ref.at[didx2], abE, sa).wait()

    def _wait_scatter(i):
        rbE, abE, sbuf, sidx, didx2, didxS, sg, sa, ss = bufs[i]
        pltpu.make_async_copy(sbuf, acc.at[didxS], ss).wait()

    def _compute_scatter(i):
        rbE, abE, sbuf, sidx, didx2, didxS, sg, sa, ss = bufs[i]
        for k in range(_EB2 // 16):
            sl = pl.ds(16 * k, 16)
            didxS[sl] = didx2[sl]

        @plsc.parallel_loop(0, _EB2, unroll=4)
        def _edge(e):
            asv = rbE[e, pl.ds(0, 16)]
            adv = abE[e, pl.ds(0, 16)]
            z = asv + adv
            al = jnp.where(z > 0, z, 0.2 * z) - cv
            exv = jnp.exp(al)
            sbuf[e, pl.ds(0, 16)] = exv
            ex0 = exv[0]
            for k in range(2):
                c0 = 16 + 16 * k
                sbuf[e, pl.ds(c0, 16)] = ex0 * rbE[e, pl.ds(c0, 16)]

        pltpu.async_copy(sbuf, acc.at[didxS], ss, add=True)

    _refill(0)
    _issue(0, 0)
    _issue(1, 1)

    def _pair(p, _):
        @pl.when(lax.rem(p + 1, (_IFILL // _EB2) // 2) == 0)
        def _():
            _refill((p + 1) // ((_IFILL // _EB2) // 2))

        _wait_gather(0)

        @pl.when(p > 0)
        def _():
            _wait_scatter(0)

        _compute_scatter(0)

        @pl.when(2 * p + 2 < _NF2)
        def _():
            _issue(0, 2 * p + 2)

        _wait_gather(1)

        @pl.when(p > 0)
        def _():
            _wait_scatter(1)

        _compute_scatter(1)

        @pl.when(2 * p + 3 < _NF2)
        def _():
            _issue(1, 2 * p + 3)

        return 0

    lax.fori_loop(0, _P2, _pair, 0)
    _wait_scatter(0)
    _wait_scatter(1)

    for k in range(_TL2 // 16):
        sl = pl.ds(16 * k, 16)
        si0[sl] = eidx[0, pl.ds(_TC2 + 16 * k, 16)]
        dSt[sl] = eidx[1, pl.ds(_TC2 + 16 * k, 16)]
    pltpu.async_copy(t2_ref.at[si0.at[pl.ds(0, _TL2)]],
                     rb0.at[pl.ds(0, _TL2)], sg0)
    pltpu.async_copy(ad2_ref.at[dSt], ab0.at[pl.ds(0, _TL2)], sa0)
    pltpu.make_async_copy(t2_ref.at[si0.at[pl.ds(0, _TL2)]],
                          rb0.at[pl.ds(0, _TL2)], sg0).wait()
    pltpu.make_async_copy(ad2_ref.at[dSt], ab0.at[pl.ds(0, _TL2)],
                          sa0).wait()

    def _tail_edge(e, _):
        asv = rb0[e, pl.ds(0, 16)]
        adv = ab0[e, pl.ds(0, 16)]
        z = asv + adv
        al = jnp.where(z > 0, z, 0.2 * z) - cv
        exv = jnp.exp(al)
        sb0[e, pl.ds(0, 16)] = exv
        ex0 = exv[0]
        for k in range(2):
            c0 = 16 + 16 * k
            sb0[e, pl.ds(c0, 16)] = ex0 * rb0[e, pl.ds(c0, 16)]
        return 0

    lax.fori_loop(0, _TL2, _tail_edge, 0)
    pltpu.async_copy(sb0.at[pl.ds(0, _TL2)], acc.at[dSt], ss0, add=True)
    pltpu.make_async_copy(sb0.at[pl.ds(0, _TL2)], acc.at[dSt], ss0).wait()
    plsc.subcore_barrier()

    for j in range((_NCHUNK + 15) // 16):
        cidx = sid + 16 * j

        @pl.when(cidx < _NCHUNK)
        def _():
            rb = cidx * _NB
            pltpu.sync_copy(acc.at[pl.ds(rb, _NB)], rb0.at[pl.ds(0, _NB)])
            pltpu.sync_copy(rb0.at[pl.ds(0, _NB)],
                            out_ref.at[cid, pl.ds(rb, _NB)])


def _sc2(edge_index, t2, ad2, cvec):
    mesh = plsc.VectorSubcoreMesh(core_axis_name="c", subcore_axis_name="s")
    f = pl.kernel(
        _sc2_body,
        out_type=jax.ShapeDtypeStruct((2, N, _G2W), jnp.float32),
        mesh=mesh,
        scratch_types=[
            pltpu.VMEM_SHARED((N, _G2W), jnp.float32),
            pltpu.VMEM((16,), jnp.float32),
            pltpu.VMEM((_EB2, _G2W), jnp.float32),
            pltpu.VMEM((_EB2, _G2W), jnp.float32),
            pltpu.VMEM((_EB2, 16), jnp.float32),
            pltpu.VMEM((_EB2, 16), jnp.float32),
            pltpu.VMEM((_EB2, _G2W), jnp.float32),
            pltpu.VMEM((_EB2, _G2W), jnp.float32),
            pltpu.VMEM((2, _IFILL), jnp.int32),
            pltpu.VMEM((_EB2,), jnp.int32),
            pltpu.VMEM((_EB2,), jnp.int32),
            pltpu.VMEM((_EB2,), jnp.int32),
            pltpu.VMEM((_EB2,), jnp.int32),
            pltpu.VMEM((_EB2,), jnp.int32),
            pltpu.VMEM((_EB2,), jnp.int32),
            pltpu.VMEM((_TL2,), jnp.int32),
            pltpu.SemaphoreType.DMA,
            pltpu.SemaphoreType.DMA,
            pltpu.SemaphoreType.DMA,
            pltpu.SemaphoreType.DMA,
            pltpu.SemaphoreType.DMA,
            pltpu.SemaphoreType.DMA,
        ],
        compiler_params=pltpu.CompilerParams(use_tc_tiling_on_sc=False),
    )
    return f(edge_index, t2, ad2, cvec)


def _dense3_body(acc_ref, bc2_ref, w2_ref, b2_ref, out_ref):
    a0 = acc_ref[0]
    a1 = acc_ref[1]
    msg = a0[:, 16:_G2W] + a1[:, 16:_G2W]
    den = jnp.broadcast_to(a0[:, 0:1] + a1[:, 0:1], (_BLK, HID))
    g2 = msg / (den + 1e-16) + bc2_ref[...][None, :]
    out_ref[...] = (
        jnp.dot(g2, w2_ref[...], preferred_element_type=jnp.float32)
        + b2_ref[...][None, :])


def _dense3(acc2, bc2, w2, b2):
    return pl.pallas_call(
        _dense3_body,
        grid=(N // _BLK,),
        in_specs=[
            pl.BlockSpec((2, _BLK, _G2W), lambda i: (0, i, 0)),
            pl.BlockSpec((HID,), lambda i: (0,)),
            pl.BlockSpec((HID, OUT_CH), lambda i: (0, 0)),
            pl.BlockSpec((OUT_CH,), lambda i: (0,)),
        ],
        out_specs=pl.BlockSpec((_BLK, OUT_CH), lambda i: (i, 0)),
        out_shape=jax.ShapeDtypeStruct((N, OUT_CH), jnp.float32),
    )(acc2, bc2, w2, b2)


def kernel(x, edge_index, W_lin1, b_lin1, Wc1, bc1, attc1_s, attc1_d,
           Wc2, bc2, attc2_s, attc2_d, W_lin2, b_lin2):
    eye = jnp.eye(HEADS, dtype=jnp.float32)
    s1 = (eye[:, None, :] * attc1_s.T[None, :, :]).reshape(HEADS * HID, HEADS)
    d1 = (eye[:, None, :] * attc1_d.T[None, :, :]).reshape(HEADS * HID, HEADS)
    s2 = jnp.pad(attc2_s.T, ((0, 0), (0, 7)))
    d2 = jnp.pad(attc2_d.T, ((0, 0), (0, 7)))

    epad = jnp.pad(edge_index, ((0, 0), (0, _EPAD - E)))

    g, adt, mxs, mxd = _dense1(x, W_lin1, b_lin1, Wc1, s1, d1)
    c1 = jnp.maximum(mxs[0] + mxd[0], 0.0)
    cvec1 = jnp.zeros((2, 16), jnp.float32).at[:, :4].set(c1.reshape(2, 4))
    acc1 = _sc1(epad, g.reshape(2 * N, _G1W),
                adt.reshape(2 * N, 16), cvec1)

    t2, ad2, mxs2, mxd2 = _dense2(acc1, bc1, Wc2, s2, d2)
    c2 = jnp.maximum(mxs2[0, 0] + mxd2[0, 0], 0.0)
    cvec2 = jnp.zeros((16,), jnp.float32).at[0].set(c2)
    acc2 = _sc2(epad, t2, ad2, cvec2)

    return _dense3(acc2, bc2, W_lin2, b_lin2)


# speedup vs baseline: 2.7359x; 1.0291x over previous
import functools

import jax
import jax.numpy as jnp
from jax import lax
from jax.experimental import pallas as pl
from jax.experimental.pallas import tpu as pltpu
from jax.experimental.pallas import tpu_sc as plsc

N = 10000
E = 320000
IN_CH = 128
HID = 32
OUT_CH = 128
HEADS = 8

_BLK = 1000
_EB = 64
_NB = 40
_NCHUNK = N // _NB

_IFILL = 512
_EPAD = E + _IFILL

_ET1 = E // 16
_NF1 = _ET1 // _EB
_P1 = _NF1 // 2
_TL1 = _ET1 - _NF1 * _EB

_EB2 = 128
_ET2 = E // 32
_NF2 = _ET2 // _EB2
_P2 = _NF2 // 2
_TL2 = _ET2 - _NF2 * _EB2
_TC2 = (_NF2 * _EB2) % _IFILL

_G1W = 144
_G2W = 48


def _dense1_body(x_ref, w1_ref, b1_ref, wc1_ref, s1_ref, d1_ref,
                 g_ref, ad_ref, mxs_ref, mxd_ref):
    i = pl.program_id(0)
    h0 = jax.nn.relu(
        jnp.dot(x_ref[...], w1_ref[...], preferred_element_type=jnp.float32)
        + b1_ref[...][None, :])
    h1 = jnp.dot(h0, wc1_ref[...], preferred_element_type=jnp.float32)
    a1s = jnp.dot(h1, s1_ref[...], preferred_element_type=jnp.float32)
    a1d = jnp.dot(h1, d1_ref[...], preferred_element_type=jnp.float32)
    z12 = jnp.zeros((_BLK, 12), jnp.float32)
    g_ref[0] = jnp.concatenate([a1s[:, 0:4], z12, h1[:, 0:128]], axis=1)
    g_ref[1] = jnp.concatenate([a1s[:, 4:8], z12, h1[:, 128:256]], axis=1)
    ad_ref[0] = jnp.concatenate([a1d[:, 0:4], z12], axis=1)
    ad_ref[1] = jnp.concatenate([a1d[:, 4:8], z12], axis=1)
    bs = jnp.max(a1s, axis=0, keepdims=True)
    bd = jnp.max(a1d, axis=0, keepdims=True)

    @pl.when(i == 0)
    def _():
        mxs_ref[...] = bs
        mxd_ref[...] = bd

    @pl.when(i > 0)
    def _():
        mxs_ref[...] = jnp.maximum(mxs_ref[...], bs)
        mxd_ref[...] = jnp.maximum(mxd_ref[...], bd)


def _dense1(x, w1, b1, wc1, s1, d1):
    return pl.pallas_call(
        _dense1_body,
        grid=(N // _BLK,),
        in_specs=[
            pl.BlockSpec((_BLK, IN_CH), lambda i: (i, 0)),
            pl.BlockSpec((IN_CH, HID), lambda i: (0, 0)),
            pl.BlockSpec((HID,), lambda i: (0,)),
            pl.BlockSpec((HID, HEADS * HID), lambda i: (0, 0)),
            pl.BlockSpec((HEADS * HID, HEADS), lambda i: (0, 0)),
            pl.BlockSpec((HEADS * HID, HEADS), lambda i: (0, 0)),
        ],
        out_specs=[
            pl.BlockSpec((2, _BLK, _G1W), lambda i: (0, i, 0)),
            pl.BlockSpec((2, _BLK, 16), lambda i: (0, i, 0)),
            pl.BlockSpec((1, HEADS), lambda i: (0, 0)),
            pl.BlockSpec((1, HEADS), lambda i: (0, 0)),
        ],
        out_shape=[
            jax.ShapeDtypeStruct((2, N, _G1W), jnp.float32),
            jax.ShapeDtypeStruct((2, N, 16), jnp.float32),
            jax.ShapeDtypeStruct((1, HEADS), jnp.float32),
            jax.ShapeDtypeStruct((1, HEADS), jnp.float32),
        ],
    )(x, w1, b1, wc1, s1, d1)


def _sc1_body(edge_ref, g_ref, adt_ref, cvec_ref, out_ref,
              acc, cbuf,
              rb0, rb1, ab0, ab1, sb0, sb1,
              eidx, si0, si1, di0, di1, dS0, dS1, dSt,
              sg0, sg1, sa0, sa1, ss0, ss1):
    cid = lax.axis_index("c")
    sid = lax.axis_index("s")
    coff = cid * N
    pltpu.sync_copy(cvec_ref, cbuf)
    cv = cbuf[cid]

    for j in range((_NCHUNK + 15) // 16):
        cidx = sid + 16 * j

        @pl.when(cidx < _NCHUNK)
        def _():
            rb = cidx * _NB
            pltpu.sync_copy(g_ref.at[pl.ds(coff + rb, _NB)],
                            rb0.at[pl.ds(0, _NB)])
            pltpu.sync_copy(adt_ref.at[pl.ds(coff + rb, _NB)],
                            ab0.at[pl.ds(0, _NB)])

            @plsc.parallel_loop(0, _NB, unroll=4)
            def _init_row(r):
                asv = rb0[r, pl.ds(0, 16)]
                adv = ab0[r, pl.ds(0, 16)]
                z = asv + adv
                al = jnp.where(z > 0, z, 0.2 * z) - cv
                exv = jnp.exp(al)
                rb0[r, pl.ds(0, 16)] = exv
                for h in range(4):
                    exh = exv[h]
                    for k in range(2):
                        c0 = 16 + 32 * h + 16 * k
                        rb0[r, pl.ds(c0, 16)] = exh * rb0[r, pl.ds(c0, 16)]
            pltpu.sync_copy(rb0.at[pl.ds(0, _NB)], acc.at[pl.ds(rb, _NB)])

    plsc.subcore_barrier()

    bufs = ((rb0, ab0, sb0, si0, di0, dS0, sg0, sa0, ss0),
            (rb1, ab1, sb1, si1, di1, dS1, sg1, sa1, ss1))
    tbase = sid * _ET1

    def _refill(q):
        pltpu.sync_copy(edge_ref.at[:, pl.ds(tbase + q * _IFILL, _IFILL)],
                        eidx)

    def _issue(i, bid):
        rbE, abE, sbuf, sidx, didx2, didxS, sg, sa, ss = bufs[i]
        m = 64 * lax.rem(bid, _IFILL // _EB)
        for k in range(_EB // 16):
            sl = pl.ds(16 * k, 16)
            sidx[sl] = eidx[0, pl.ds(m + 16 * k, 16)] + coff
            didx2[sl] = eidx[1, pl.ds(m + 16 * k, 16)] + coff
        pltpu.async_copy(g_ref.at[sidx], rbE, sg)
        pltpu.async_copy(adt_ref.at[didx2], abE, sa)

    def _wait_gather(i):
        rbE, abE, sbuf, sidx, didx2, didxS, sg, sa, ss = bufs[i]
        pltpu.make_async_copy(g_ref.at[sidx], rbE, sg).wait()
        pltpu.make_async_copy(adt_ref.at[didx2], abE, sa).wait()

    def _wait_scatter(i):
        rbE, abE, sbuf, sidx, didx2, didxS, sg, sa, ss = bufs[i]
        pltpu.make_async_copy(sbuf, acc.at[didxS], ss).wait()

    def _compute_scatter(i):
        rbE, abE, sbuf, sidx, didx2, didxS, sg, sa, ss = bufs[i]
        for k in range(_EB // 16):
            sl = pl.ds(16 * k, 16)
            didxS[sl] = didx2[sl] - coff

        @plsc.parallel_loop(0, _EB, unroll=8)
        def _edge(e):
            asv = rbE[e, pl.ds(0, 16)]
            adv = abE[e, pl.ds(0, 16)]
            z = asv + adv
            al = jnp.where(z > 0, z, 0.2 * z) - cv
            exv = jnp.exp(al)
            sbuf[e, pl.ds(0, 16)] = exv
            for h in range(4):
                exh = exv[h]
                for k in range(2):
                    c0 = 16 + 32 * h + 16 * k
                    sbuf[e, pl.ds(c0, 16)] = exh * rbE[e, pl.ds(c0, 16)]

        pltpu.async_copy(sbuf, acc.at[didxS], ss, add=True)

    _refill(0)
    _issue(0, 0)
    _issue(1, 1)

    def _pair(p, _):
        @pl.when(lax.rem(p + 1, (_IFILL // _EB) // 2) == 0)
        def _():
            _refill((p + 1) // ((_IFILL // _EB) // 2))

        _wait_gather(0)

        @pl.when(p > 0)
        def _():
            _wait_scatter(0)

        _compute_scatter(0)

        @pl.when(2 * p + 2 < _NF1)
        def _():
            _issue(0, 2 * p + 2)

        _wait_gather(1)

        @pl.when(p > 0)
        def _():
            _wait_scatter(1)

        _compute_scatter(1)

        @pl.when(2 * p + 3 < _NF1)
        def _():
            _issue(1, 2 * p + 3)

        return 0

    lax.fori_loop(0, _P1, _pair, 0)
    _wait_scatter(0)
    _wait_scatter(1)

    for k in range(_TL1 // 16):
        sl = pl.ds(16 * k, 16)
        si0[sl] = eidx[0, pl.ds(16 * k, 16)] + coff
        di0[sl] = eidx[1, pl.ds(16 * k, 16)] + coff
        dSt[sl] = eidx[1, pl.ds(16 * k, 16)]
    pltpu.async_copy(g_ref.at[si0.at[pl.ds(0, _TL1)]],
                     rb0.at[pl.ds(0, _TL1)], sg0)
    pltpu.async_copy(adt_ref.at[di0.at[pl.ds(0, _TL1)]],
                     ab0.at[pl.ds(0, _TL1)], sa0)
    pltpu.make_async_copy(g_ref.at[si0.at[pl.ds(0, _TL1)]],
                          rb0.at[pl.ds(0, _TL1)], sg0).wait()
    pltpu.make_async_copy(adt_ref.at[di0.at[pl.ds(0, _TL1)]],
                          ab0.at[pl.ds(0, _TL1)], sa0).wait()

    def _tail_edge(e, _):
        asv = rb0[e, pl.ds(0, 16)]
        adv = ab0[e, pl.ds(0, 16)]
        z = asv + adv
        al = jnp.where(z > 0, z, 0.2 * z) - cv
        exv = jnp.exp(al)
        sb0[e, pl.ds(0, 16)] = exv
        for h in range(4):
            exh = exv[h]
            for k in range(2):
                c0 = 16 + 32 * h + 16 * k
                sb0[e, pl.ds(c0, 16)] = exh * rb0[e, pl.ds(c0, 16)]
        return 0

    lax.fori_loop(0, _TL1, _tail_edge, 0)
    pltpu.async_copy(sb0.at[pl.ds(0, _TL1)],
                     acc.at[dSt], ss0, add=True)
    pltpu.make_async_copy(sb0.at[pl.ds(0, _TL1)],
                          acc.at[dSt], ss0).wait()
    plsc.subcore_barrier()

    for j in range((_NCHUNK + 15) // 16):
        cidx = sid + 16 * j

        @pl.when(cidx < _NCHUNK)
        def _():
            rb = cidx * _NB
            pltpu.sync_copy(acc.at[pl.ds(rb, _NB)], rb0.at[pl.ds(0, _NB)])
            pltpu.sync_copy(rb0.at[pl.ds(0, _NB)],
                            out_ref.at[cid, pl.ds(rb, _NB)])


def _sc1(edge_index, g, adt, cvec):
    mesh = plsc.VectorSubcoreMesh(core_axis_name="c", subcore_axis_name="s")
    f = pl.kernel(
        _sc1_body,
        out_type=jax.ShapeDtypeStruct((2, N, _G1W), jnp.float32),
        mesh=mesh,
        scratch_types=[
            pltpu.VMEM_SHARED((N, _G1W), jnp.float32),
            pltpu.VMEM((2, 16), jnp.float32),
            pltpu.VMEM((_EB, _G1W), jnp.float32),
            pltpu.VMEM((_EB, _G1W), jnp.float32),
            pltpu.VMEM((_EB, 16), jnp.float32),
            pltpu.VMEM((_EB, 16), jnp.float32),
            pltpu.VMEM((_EB, _G1W), jnp.float32),
            pltpu.VMEM((_EB, _G1W), jnp.float32),
            pltpu.VMEM((2, _IFILL), jnp.int32),
            pltpu.VMEM((_EB,), jnp.int32),
            pltpu.VMEM((_EB,), jnp.int32),
            pltpu.VMEM((_EB,), jnp.int32),
            pltpu.VMEM((_EB,), jnp.int32),
            pltpu.VMEM((_EB,), jnp.int32),
            pltpu.VMEM((_EB,), jnp.int32),
            pltpu.VMEM((_TL1,), jnp.int32),
            pltpu.SemaphoreType.DMA,
            pltpu.SemaphoreType.DMA,
            pltpu.SemaphoreType.DMA,
            pltpu.SemaphoreType.DMA,
            pltpu.SemaphoreType.DMA,
            pltpu.SemaphoreType.DMA,
        ],
        compiler_params=pltpu.CompilerParams(use_tc_tiling_on_sc=False),
    )
    return f(edge_index, g, adt, cvec)


def _dense2_body(acc_ref, bc1_ref, wc2_ref, s2_ref, d2_ref,
                 t2_ref, ad2_ref, mxs_ref, mxd_ref):
    i = pl.program_id(0)
    a0 = acc_ref[0]
    a1 = acc_ref[1]
    msg = jnp.concatenate([a0[:, 16:_G1W], a1[:, 16:_G1W]], axis=1)
    dens = []
    for c in range(2):
        a = a0 if c == 0 else a1
        for h in range(4):
            dens.append(jnp.broadcast_to(a[:, h:h + 1], (_BLK, HID)))
    den = jnp.concatenate(dens, axis=1)
    g1 = msg / (den + 1e-16) + bc1_ref[...][None, :]
    g1 = jnp.where(g1 > 0, g1, jnp.exp(g1) - 1.0)
    h2 = jnp.dot(g1, wc2_ref[...], preferred_element_type=jnp.float32)
    a2s = jnp.dot(h2, s2_ref[...], preferred_element_type=jnp.float32)
    a2d = jnp.dot(h2, d2_ref[...], preferred_element_type=jnp.float32)
    z15 = jnp.zeros((_BLK, 15), jnp.float32)
    t2_ref[...] = jnp.concatenate([a2s[:, 0:1], z15, h2], axis=1)
    ad2_ref[...] = jnp.concatenate([a2d[:, 0:1], z15], axis=1)
    bs = jnp.max(a2s, axis=0, keepdims=True)
    bd = jnp.max(a2d, axis=0, keepdims=True)

    @pl.when(i == 0)
    def _():
        mxs_ref[...] = bs
        mxd_ref[...] = bd

    @pl.when(i > 0)
    def _():
        mxs_ref[...] = jnp.maximum(mxs_ref[...], bs)
        mxd_ref[...] = jnp.maximum(mxd_ref[...], bd)


def _dense2(acc1, bc1, wc2, s2, d2):
    return pl.pallas_call(
        _dense2_body,
        grid=(N // _BLK,),
        in_specs=[
            pl.BlockSpec((2, _BLK, _G1W), lambda i: (0, i, 0)),
            pl.BlockSpec((HEADS * HID,), lambda i: (0,)),
            pl.BlockSpec((HEADS * HID, HID), lambda i: (0, 0)),
            pl.BlockSpec((HID, 8), lambda i: (0, 0)),
            pl.BlockSpec((HID, 8), lambda i: (0, 0)),
        ],
        out_specs=[
            pl.BlockSpec((_BLK, _G2W), lambda i: (i, 0)),
            pl.BlockSpec((_BLK, 16), lambda i: (i, 0)),
            pl.BlockSpec((1, 8), lambda i: (0, 0)),
            pl.BlockSpec((1, 8), lambda i: (0, 0)),
        ],
        out_shape=[
            jax.ShapeDtypeStruct((N, _G2W), jnp.float32),
            jax.ShapeDtypeStruct((N, 16), jnp.float32),
            jax.ShapeDtypeStruct((1, 8), jnp.float32),
            jax.ShapeDtypeStruct((1, 8), jnp.float32),
        ],
    )(acc1, bc1, wc2, s2, d2)


def _sc2_body(edge_ref, t2_ref, ad2_ref, cvec_ref, out_ref,
              acc, cbuf,
              rb0, rb1, ab0, ab1, sb0, sb1,
              eidx, si0, si1, di0, di1, dS0, dS1, dSt,
              sg0, sg1, sa0, sa1, ss0, ss1):
    cid = lax.axis_index("c")
    sid = lax.axis_index("s")
    pltpu.sync_copy(cvec_ref, cbuf)
    cv = cbuf[...]
    scale = jnp.where(cid == 0, 1.0, 0.0)

    for j in range((_NCHUNK + 15) // 16):
        cidx = sid + 16 * j

        @pl.when(cidx < _NCHUNK)
        def _():
            rb = cidx * _NB
            pltpu.sync_copy(t2_ref.at[pl.ds(rb, _NB)], rb0.at[pl.ds(0, _NB)])
            pltpu.sync_copy(ad2_ref.at[pl.ds(rb, _NB)], ab0.at[pl.ds(0, _NB)])

            @plsc.parallel_loop(0, _NB, unroll=4)
            def _init_row(r):
                asv = rb0[r, pl.ds(0, 16)]
                adv = ab0[r, pl.ds(0, 16)]
                z = asv + adv
                al = jnp.where(z > 0, z, 0.2 * z) - cv
                exv = jnp.exp(al) * scale
                rb0[r, pl.ds(0, 16)] = exv
                ex0 = exv[0]
                for k in range(2):
                    c0 = 16 + 16 * k
                    rb0[r, pl.ds(c0, 16)] = ex0 * rb0[r, pl.ds(c0, 16)]
            pltpu.sync_copy(rb0.at[pl.ds(0, _NB)], acc.at[pl.ds(rb, _NB)])

    plsc.subcore_barrier()

    bufs = ((rb0, ab0, sb0, si0, di0, dS0, sg0, sa0, ss0),
            (rb1, ab1, sb1, si1, di1, dS1, sg1, sa1, ss1))
    tbase = cid * (E // 2) + sid * _ET2

    def _refill(q):
        pltpu.sync_copy(edge_ref.at[:, pl.ds(tbase + q * _IFILL, _IFILL)],
                        eidx)

    def _issue(i, bid):
        rbE, abE, sbuf, sidx, didx2, didxS, sg, sa, ss = bufs[i]
        m = _EB2 * lax.rem(bid, _IFILL // _EB2)
        for k in range(_EB2 // 16):
            sl = pl.ds(16 * k, 16)
            sidx[sl] = eidx[0, pl.ds(m + 16 * k, 16)]
            didx2[sl] = eidx[1, pl.ds(m + 16 * k, 16)]
        pltpu.async_copy(t2_ref.at[sidx], rbE, sg)
        pltpu.async_copy(ad2_ref.at[didx2], abE, sa)

    def _wait_gather(i):
        rbE, abE, sbuf, sidx, didx2, didxS, sg, sa, ss = bufs[i]
        pltpu.make_async_copy(t2_ref.at[sidx], rbE, sg).wait()
        pltpu.make_async_copy(ad2_ref.at[didx2], abE, sa).wait()

    def _wait_scatter(i):
        rbE, abE, sbuf, sidx, didx2, didxS, sg, sa, ss = bufs[i]
        pltpu.make_async_copy(sbuf, acc.at[didxS], ss).wait()

    def _compute_scatter(i):
        rbE, abE, sbuf, sidx, didx2, didxS, sg, sa, ss = bufs[i]
        for k in range(_EB2 // 16):
            sl = pl.ds(16 * k, 16)
            didxS[sl] = didx2[sl]

        @plsc.parallel_loop(0, _EB2, unroll=8)
        def _edge(e):
            asv = rbE[e, pl.ds(0, 16)]
            adv = abE[e, pl.ds(0, 16)]
            z = asv + adv
            al = jnp.where(z > 0, z, 0.2 * z) - cv
            exv = jnp.exp(al)
            sbuf[e, pl.ds(0, 16)] = exv
            ex0 = exv[0]
            for k in range(2):
                c0 = 16 + 16 * k
                sbuf[e, pl.ds(c0, 16)] = ex0 * rbE[e, pl.ds(c0, 16)]

        pltpu.async_copy(sbuf, acc.at[didxS], ss, add=True)

    _refill(0)
    _issue(0, 0)
    _issue(1, 1)

    def _pair(p, _):
        @pl.when(lax.rem(p + 1, (_IFILL // _EB2) // 2) == 0)
        def _():
            _refill((p + 1) // ((_IFILL // _EB2) // 2))

        _wait_gather(0)

        @pl.when(p > 0)
        def _():
            _wait_scatter(0)

        _compute_scatter(0)

        @pl.when(2 * p + 2 < _NF2)
        def _():
            _issue(0, 2 * p + 2)

        _wait_gather(1)

        @pl.when(p > 0)
        def _():
            _wait_scatter(1)

        _compute_scatter(1)

        @pl.when(2 * p + 3 < _NF2)
        def _():
            _issue(1, 2 * p + 3)

        return 0

    lax.fori_loop(0, _P2, _pair, 0)
    _wait_scatter(0)
    _wait_scatter(1)

    for k in range(_TL2 // 16):
        sl = pl.ds(16 * k, 16)
        si0[sl] = eidx[0, pl.ds(_TC2 + 16 * k, 16)]
        dSt[sl] = eidx[1, pl.ds(_TC2 + 16 * k, 16)]
    pltpu.async_copy(t2_ref.at[si0.at[pl.ds(0, _TL2)]],
                     rb0.at[pl.ds(0, _TL2)], sg0)
    pltpu.async_copy(ad2_ref.at[dSt], ab0.at[pl.ds(0, _TL2)], sa0)
    pltpu.make_async_copy(t2_ref.at[si0.at[pl.ds(0, _TL2)]],
                          rb0.at[pl.ds(0, _TL2)], sg0).wait()
    pltpu.make_async_copy(ad2_ref.at[dSt], ab0.at[pl.ds(0, _TL2)],
                          sa0).wait()

    def _tail_edge(e, _):
        asv = rb0[e, pl.ds(0, 16)]
        adv = ab0[e, pl.ds(0, 16)]
        z = asv + adv
        al = jnp.where(z > 0, z, 0.2 * z) - cv
        exv = jnp.exp(al)
        sb0[e, pl.ds(0, 16)] = exv
        ex0 = exv[0]
        for k in range(2):
            c0 = 16 + 16 * k
            sb0[e, pl.ds(c0, 16)] = ex0 * rb0[e, pl.ds(c0, 16)]
        return 0

    lax.fori_loop(0, _TL2, _tail_edge, 0)
    pltpu.async_copy(sb0.at[pl.ds(0, _TL2)], acc.at[dSt], ss0, add=True)
    pltpu.make_async_copy(sb0.at[pl.ds(0, _TL2)], acc.at[dSt], ss0).wait()
    plsc.subcore_barrier()

    for j in range((_NCHUNK + 15) // 16):
        cidx = sid + 16 * j

        @pl.when(cidx < _NCHUNK)
        def _():
            rb = cidx * _NB
            pltpu.sync_copy(acc.at[pl.ds(rb, _NB)], rb0.at[pl.ds(0, _NB)])
            pltpu.sync_copy(rb0.at[pl.ds(0, _NB)],
                            out_ref.at[cid, pl.ds(rb, _NB)])


def _sc2(edge_index, t2, ad2, cvec):
    mesh = plsc.VectorSubcoreMesh(core_axis_name="c", subcore_axis_name="s")
    f = pl.kernel(
        _sc2_body,
        out_type=jax.ShapeDtypeStruct((2, N, _G2W), jnp.float32),
        mesh=mesh,
        scratch_types=[
            pltpu.VMEM_SHARED((N, _G2W), jnp.float32),
            pltpu.VMEM((16,), jnp.float32),
            pltpu.VMEM((_EB2, _G2W), jnp.float32),
            pltpu.VMEM((_EB2, _G2W), jnp.float32),
            pltpu.VMEM((_EB2, 16), jnp.float32),
            pltpu.VMEM((_EB2, 16), jnp.float32),
            pltpu.VMEM((_EB2, _G2W), jnp.float32),
            pltpu.VMEM((_EB2, _G2W), jnp.float32),
            pltpu.VMEM((2, _IFILL), jnp.int32),
            pltpu.VMEM((_EB2,), jnp.int32),
            pltpu.VMEM((_EB2,), jnp.int32),
            pltpu.VMEM((_EB2,), jnp.int32),
            pltpu.VMEM((_EB2,), jnp.int32),
            pltpu.VMEM((_EB2,), jnp.int32),
            pltpu.VMEM((_EB2,), jnp.int32),
            pltpu.VMEM((_TL2,), jnp.int32),
            pltpu.SemaphoreType.DMA,
            pltpu.SemaphoreType.DMA,
            pltpu.SemaphoreType.DMA,
            pltpu.SemaphoreType.DMA,
            pltpu.SemaphoreType.DMA,
            pltpu.SemaphoreType.DMA,
        ],
        compiler_params=pltpu.CompilerParams(use_tc_tiling_on_sc=False),
    )
    return f(edge_index, t2, ad2, cvec)


def _dense3_body(acc_ref, bc2_ref, w2_ref, b2_ref, out_ref):
    a0 = acc_ref[0]
    a1 = acc_ref[1]
    msg = a0[:, 16:_G2W] + a1[:, 16:_G2W]
    den = jnp.broadcast_to(a0[:, 0:1] + a1[:, 0:1], (_BLK, HID))
    g2 = msg / (den + 1e-16) + bc2_ref[...][None, :]
    out_ref[...] = (
        jnp.dot(g2, w2_ref[...], preferred_element_type=jnp.float32)
        + b2_ref[...][None, :])


def _dense3(acc2, bc2, w2, b2):
    return pl.pallas_call(
        _dense3_body,
        grid=(N // _BLK,),
        in_specs=[
            pl.BlockSpec((2, _BLK, _G2W), lambda i: (0, i, 0)),
            pl.BlockSpec((HID,), lambda i: (0,)),
            pl.BlockSpec((HID, OUT_CH), lambda i: (0, 0)),
            pl.BlockSpec((OUT_CH,), lambda i: (0,)),
        ],
        out_specs=pl.BlockSpec((_BLK, OUT_CH), lambda i: (i, 0)),
        out_shape=jax.ShapeDtypeStruct((N, OUT_CH), jnp.float32),
    )(acc2, bc2, w2, b2)


def kernel(x, edge_index, W_lin1, b_lin1, Wc1, bc1, attc1_s, attc1_d,
           Wc2, bc2, attc2_s, attc2_d, W_lin2, b_lin2):
    eye = jnp.eye(HEADS, dtype=jnp.float32)
    s1 = (eye[:, None, :] * attc1_s.T[None, :, :]).reshape(HEADS * HID, HEADS)
    d1 = (eye[:, None, :] * attc1_d.T[None, :, :]).reshape(HEADS * HID, HEADS)
    s2 = jnp.pad(attc2_s.T, ((0, 0), (0, 7)))
    d2 = jnp.pad(attc2_d.T, ((0, 0), (0, 7)))

    epad = jnp.pad(edge_index, ((0, 0), (0, _EPAD - E)))

    g, adt, mxs, mxd = _dense1(x, W_lin1, b_lin1, Wc1, s1, d1)
    c1 = jnp.maximum(mxs[0] + mxd[0], 0.0)
    cvec1 = jnp.zeros((2, 16), jnp.float32).at[:, :4].set(c1.reshape(2, 4))
    acc1 = _sc1(epad, g.reshape(2 * N, _G1W),
                adt.reshape(2 * N, 16), cvec1)

    t2, ad2, mxs2, mxd2 = _dense2(acc1, bc1, Wc2, s2, d2)
    c2 = jnp.maximum(mxs2[0, 0] + mxd2[0, 0], 0.0)
    cvec2 = jnp.zeros((16,), jnp.float32).at[0].set(c2)
    acc2 = _sc2(epad, t2, ad2, cvec2)

    return _dense3(acc2, bc2, W_lin2, b_lin2)


# speedup vs baseline: 2.7700x; 1.0125x over previous
import functools

import jax
import jax.numpy as jnp
from jax import lax
from jax.experimental import pallas as pl
from jax.experimental.pallas import tpu as pltpu
from jax.experimental.pallas import tpu_sc as plsc

N = 10000
E = 320000
IN_CH = 128
HID = 32
OUT_CH = 128
HEADS = 8

_BLK = 2000
_EB = 64
_NB = 40
_NCHUNK = N // _NB

_IFILL = 512
_EPAD = E + _IFILL

_ET1 = E // 16
_NF1 = _ET1 // _EB
_P1 = _NF1 // 2
_TL1 = _ET1 - _NF1 * _EB

_EB2 = 128
_ET2 = E // 32
_NF2 = _ET2 // _EB2
_P2 = _NF2 // 2
_TL2 = _ET2 - _NF2 * _EB2
_TC2 = (_NF2 * _EB2) % _IFILL

_G1W = 144
_G2W = 48


def _dense1_body(x_ref, w1_ref, b1_ref, wc1_ref, s1_ref, d1_ref,
                 g_ref, ad_ref, mxs_ref, mxd_ref):
    i = pl.program_id(0)
    h0 = jax.nn.relu(
        jnp.dot(x_ref[...], w1_ref[...], preferred_element_type=jnp.float32)
        + b1_ref[...][None, :])
    h1 = jnp.dot(h0, wc1_ref[...], preferred_element_type=jnp.float32)
    a1s = jnp.dot(h1, s1_ref[...], preferred_element_type=jnp.float32)
    a1d = jnp.dot(h1, d1_ref[...], preferred_element_type=jnp.float32)
    z12 = jnp.zeros((_BLK, 12), jnp.float32)
    g_ref[0] = jnp.concatenate([a1s[:, 0:4], z12, h1[:, 0:128]], axis=1)
    g_ref[1] = jnp.concatenate([a1s[:, 4:8], z12, h1[:, 128:256]], axis=1)
    ad_ref[0] = jnp.concatenate([a1d[:, 0:4], z12], axis=1)
    ad_ref[1] = jnp.concatenate([a1d[:, 4:8], z12], axis=1)
    bs = jnp.max(a1s, axis=0, keepdims=True)
    bd = jnp.max(a1d, axis=0, keepdims=True)

    @pl.when(i == 0)
    def _():
        mxs_ref[...] = bs
        mxd_ref[...] = bd

    @pl.when(i > 0)
    def _():
        mxs_ref[...] = jnp.maximum(mxs_ref[...], bs)
        mxd_ref[...] = jnp.maximum(mxd_ref[...], bd)


def _dense1(x, w1, b1, wc1, s1, d1):
    return pl.pallas_call(
        _dense1_body,
        grid=(N // _BLK,),
        in_specs=[
            pl.BlockSpec((_BLK, IN_CH), lambda i: (i, 0)),
            pl.BlockSpec((IN_CH, HID), lambda i: (0, 0)),
            pl.BlockSpec((HID,), lambda i: (0,)),
            pl.BlockSpec((HID, HEADS * HID), lambda i: (0, 0)),
            pl.BlockSpec((HEADS * HID, HEADS), lambda i: (0, 0)),
            pl.BlockSpec((HEADS * HID, HEADS), lambda i: (0, 0)),
        ],
        out_specs=[
            pl.BlockSpec((2, _BLK, _G1W), lambda i: (0, i, 0)),
            pl.BlockSpec((2, _BLK, 16), lambda i: (0, i, 0)),
            pl.BlockSpec((1, HEADS), lambda i: (0, 0)),
            pl.BlockSpec((1, HEADS), lambda i: (0, 0)),
        ],
        out_shape=[
            jax.ShapeDtypeStruct((2, N, _G1W), jnp.float32),
            jax.ShapeDtypeStruct((2, N, 16), jnp.float32),
            jax.ShapeDtypeStruct((1, HEADS), jnp.float32),
            jax.ShapeDtypeStruct((1, HEADS), jnp.float32),
        ],
    )(x, w1, b1, wc1, s1, d1)


def _sc1_body(edge_ref, g_ref, adt_ref, cvec_ref, out_ref,
              acc, cbuf,
              rb0, rb1, ab0, ab1, sb0, sb1,
              eidx, si0, si1, di0, di1, dS0, dS1, dSt,
              sg0, sg1, sa0, sa1, ss0, ss1):
    cid = lax.axis_index("c")
    sid = lax.axis_index("s")
    coff = cid * N
    pltpu.sync_copy(cvec_ref, cbuf)
    cv = cbuf[cid]

    for j in range((_NCHUNK + 15) // 16):
        cidx = sid + 16 * j

        @pl.when(cidx < _NCHUNK)
        def _():
            rb = cidx * _NB
            pltpu.sync_copy(g_ref.at[pl.ds(coff + rb, _NB)],
                            rb0.at[pl.ds(0, _NB)])
            pltpu.sync_copy(adt_ref.at[pl.ds(coff + rb, _NB)],
                            ab0.at[pl.ds(0, _NB)])

            @plsc.parallel_loop(0, _NB, unroll=4)
            def _init_row(r):
                asv = rb0[r, pl.ds(0, 16)]
                adv = ab0[r, pl.ds(0, 16)]
                z = asv + adv
                al = jnp.where(z > 0, z, 0.2 * z) - cv
                exv = jnp.exp(al)
                rb0[r, pl.ds(0, 16)] = exv
                for h in range(4):
                    exh = exv[h]
                    for k in range(2):
                        c0 = 16 + 32 * h + 16 * k
                        rb0[r, pl.ds(c0, 16)] = exh * rb0[r, pl.ds(c0, 16)]
            pltpu.sync_copy(rb0.at[pl.ds(0, _NB)], acc.at[pl.ds(rb, _NB)])

    plsc.subcore_barrier()

    bufs = ((rb0, ab0, sb0, si0, di0, dS0, sg0, sa0, ss0),
            (rb1, ab1, sb1, si1, di1, dS1, sg1, sa1, ss1))
    tbase = sid * _ET1

    def _refill(q):
        pltpu.sync_copy(edge_ref.at[:, pl.ds(tbase + q * _IFILL, _IFILL)],
                        eidx)

    def _issue(i, bid):
        rbE, abE, sbuf, sidx, didx2, didxS, sg, sa, ss = bufs[i]
        m = 64 * lax.rem(bid, _IFILL // _EB)
        for k in range(_EB // 16):
            sl = pl.ds(16 * k, 16)
            sidx[sl] = eidx[0, pl.ds(m + 16 * k, 16)] + coff
            didx2[sl] = eidx[1, pl.ds(m + 16 * k, 16)] + coff
        pltpu.async_copy(g_ref.at[sidx], rbE, sg)
        pltpu.async_copy(adt_ref.at[didx2], abE, sa)

    def _wait_gather(i):
        rbE, abE, sbuf, sidx, didx2, didxS, sg, sa, ss = bufs[i]
        pltpu.make_async_copy(g_ref.at[sidx], rbE, sg).wait()
        pltpu.make_async_copy(adt_ref.at[didx2], abE, sa).wait()

    def _wait_scatter(i):
        rbE, abE, sbuf, sidx, didx2, didxS, sg, sa, ss = bufs[i]
        pltpu.make_async_copy(sbuf, acc.at[didxS], ss).wait()

    def _compute_scatter(i):
        rbE, abE, sbuf, sidx, didx2, didxS, sg, sa, ss = bufs[i]
        for k in range(_EB // 16):
            sl = pl.ds(16 * k, 16)
            didxS[sl] = didx2[sl] - coff

        @plsc.parallel_loop(0, _EB, unroll=8)
        def _edge(e):
            asv = rbE[e, pl.ds(0, 16)]
            adv = abE[e, pl.ds(0, 16)]
            z = asv + adv
            al = jnp.where(z > 0, z, 0.2 * z) - cv
            exv = jnp.exp(al)
            sbuf[e, pl.ds(0, 16)] = exv
            for h in range(4):
                exh = exv[h]
                for k in range(2):
                    c0 = 16 + 32 * h + 16 * k
                    sbuf[e, pl.ds(c0, 16)] = exh * rbE[e, pl.ds(c0, 16)]

        pltpu.async_copy(sbuf, acc.at[didxS], ss, add=True)

    _refill(0)
    _issue(0, 0)
    _issue(1, 1)

    def _pair(p, _):
        @pl.when(lax.rem(p + 1, (_IFILL // _EB) // 2) == 0)
        def _():
            _refill((p + 1) // ((_IFILL // _EB) // 2))

        _wait_gather(0)

        @pl.when(p > 0)
        def _():
            _wait_scatter(0)

        _compute_scatter(0)

        @pl.when(2 * p + 2 < _NF1)
        def _():
            _issue(0, 2 * p + 2)

        _wait_gather(1)

        @pl.when(p > 0)
        def _():
            _wait_scatter(1)

        _compute_scatter(1)

        @pl.when(2 * p + 3 < _NF1)
        def _():
            _issue(1, 2 * p + 3)

        return 0

    lax.fori_loop(0, _P1, _pair, 0)
    _wait_scatter(0)
    _wait_scatter(1)

    for k in range(_TL1 // 16):
        sl = pl.ds(16 * k, 16)
        si0[sl] = eidx[0, pl.ds(16 * k, 16)] + coff
        di0[sl] = eidx[1, pl.ds(16 * k, 16)] + coff
        dSt[sl] = eidx[1, pl.ds(16 * k, 16)]
    pltpu.async_copy(g_ref.at[si0.at[pl.ds(0, _TL1)]],
                     rb0.at[pl.ds(0, _TL1)], sg0)
    pltpu.async_copy(adt_ref.at[di0.at[pl.ds(0, _TL1)]],
                     ab0.at[pl.ds(0, _TL1)], sa0)
    pltpu.make_async_copy(g_ref.at[si0.at[pl.ds(0, _TL1)]],
                          rb0.at[pl.ds(0, _TL1)], sg0).wait()
    pltpu.make_async_copy(adt_ref.at[di0.at[pl.ds(0, _TL1)]],
                          ab0.at[pl.ds(0, _TL1)], sa0).wait()

    def _tail_edge(e, _):
        asv = rb0[e, pl.ds(0, 16)]
        adv = ab0[e, pl.ds(0, 16)]
        z = asv + adv
        al = jnp.where(z > 0, z, 0.2 * z) - cv
        exv = jnp.exp(al)
        sb0[e, pl.ds(0, 16)] = exv
        for h in range(4):
            exh = exv[h]
            for k in range(2):
                c0 = 16 + 32 * h + 16 * k
                sb0[e, pl.ds(c0, 16)] = exh * rb0[e, pl.ds(c0, 16)]
        return 0

    lax.fori_loop(0, _TL1, _tail_edge, 0)
    pltpu.async_copy(sb0.at[pl.ds(0, _TL1)],
                     acc.at[dSt], ss0, add=True)
    pltpu.make_async_copy(sb0.at[pl.ds(0, _TL1)],
                          acc.at[dSt], ss0).wait()
    plsc.subcore_barrier()

    for j in range((_NCHUNK + 15) // 16):
        cidx = sid + 16 * j

        @pl.when(cidx < _NCHUNK)
        def _():
            rb = cidx * _NB
            pltpu.sync_copy(acc.at[pl.ds(rb, _NB)], rb0.at[pl.ds(0, _NB)])
            pltpu.sync_copy(rb0.at[pl.ds(0, _NB)],
                            out_ref.at[cid, pl.ds(rb, _NB)])


def _sc1(edge_index, g, adt, cvec):
    mesh = plsc.VectorSubcoreMesh(core_axis_name="c", subcore_axis_name="s")
    f = pl.kernel(
        _sc1_body,
        out_type=jax.ShapeDtypeStruct((2, N, _G1W), jnp.float32),
        mesh=mesh,
        scratch_types=[
            pltpu.VMEM_SHARED((N, _G1W), jnp.float32),
            pltpu.VMEM((2, 16), jnp.float32),
            pltpu.VMEM((_EB, _G1W), jnp.float32),
            pltpu.VMEM((_EB, _G1W), jnp.float32),
            pltpu.VMEM((_EB, 16), jnp.float32),
            pltpu.VMEM((_EB, 16), jnp.float32),
            pltpu.VMEM((_EB, _G1W), jnp.float32),
            pltpu.VMEM((_EB, _G1W), jnp.float32),
            pltpu.VMEM((2, _IFILL), jnp.int32),
            pltpu.VMEM((_EB,), jnp.int32),
            pltpu.VMEM((_EB,), jnp.int32),
            pltpu.VMEM((_EB,), jnp.int32),
            pltpu.VMEM((_EB,), jnp.int32),
            pltpu.VMEM((_EB,), jnp.int32),
            pltpu.VMEM((_EB,), jnp.int32),
            pltpu.VMEM((_TL1,), jnp.int32),
            pltpu.SemaphoreType.DMA,
            pltpu.SemaphoreType.DMA,
            pltpu.SemaphoreType.DMA,
            pltpu.SemaphoreType.DMA,
            pltpu.SemaphoreType.DMA,
            pltpu.SemaphoreType.DMA,
        ],
        compiler_params=pltpu.CompilerParams(use_tc_tiling_on_sc=False),
    )
    return f(edge_index, g, adt, cvec)


def _dense2_body(acc_ref, bc1_ref, wc2_ref, s2_ref, d2_ref,
                 t2_ref, ad2_ref, mxs_ref, mxd_ref):
    i = pl.program_id(0)
    a0 = acc_ref[0]
    a1 = acc_ref[1]
    msg = jnp.concatenate([a0[:, 16:_G1W], a1[:, 16:_G1W]], axis=1)
    dens = []
    for c in range(2):
        a = a0 if c == 0 else a1
        for h in range(4):
            dens.append(jnp.broadcast_to(a[:, h:h + 1], (_BLK, HID)))
    den = jnp.concatenate(dens, axis=1)
    g1 = msg / (den + 1e-16) + bc1_ref[...][None, :]
    g1 = jnp.where(g1 > 0, g1, jnp.exp(g1) - 1.0)
    h2 = jnp.dot(g1, wc2_ref[...], preferred_element_type=jnp.float32)
    a2s = jnp.dot(h2, s2_ref[...], preferred_element_type=jnp.float32)
    a2d = jnp.dot(h2, d2_ref[...], preferred_element_type=jnp.float32)
    z15 = jnp.zeros((_BLK, 15), jnp.float32)
    t2_ref[...] = jnp.concatenate([a2s[:, 0:1], z15, h2], axis=1)
    ad2_ref[...] = jnp.concatenate([a2d[:, 0:1], z15], axis=1)
    bs = jnp.max(a2s, axis=0, keepdims=True)
    bd = jnp.max(a2d, axis=0, keepdims=True)

    @pl.when(i == 0)
    def _():
        mxs_ref[...] = bs
        mxd_ref[...] = bd

    @pl.when(i > 0)
    def _():
        mxs_ref[...] = jnp.maximum(mxs_ref[...], bs)
        mxd_ref[...] = jnp.maximum(mxd_ref[...], bd)


def _dense2(acc1, bc1, wc2, s2, d2):
    return pl.pallas_call(
        _dense2_body,
        grid=(N // _BLK,),
        in_specs=[
            pl.BlockSpec((2, _BLK, _G1W), lambda i: (0, i, 0)),
            pl.BlockSpec((HEADS * HID,), lambda i: (0,)),
            pl.BlockSpec((HEADS * HID, HID), lambda i: (0, 0)),
            pl.BlockSpec((HID, 8), lambda i: (0, 0)),
            pl.BlockSpec((HID, 8), lambda i: (0, 0)),
        ],
        out_specs=[
            pl.BlockSpec((_BLK, _G2W), lambda i: (i, 0)),
            pl.BlockSpec((_BLK, 16), lambda i: (i, 0)),
            pl.BlockSpec((1, 8), lambda i: (0, 0)),
            pl.BlockSpec((1, 8), lambda i: (0, 0)),
        ],
        out_shape=[
            jax.ShapeDtypeStruct((N, _G2W), jnp.float32),
            jax.ShapeDtypeStruct((N, 16), jnp.float32),
            jax.ShapeDtypeStruct((1, 8), jnp.float32),
            jax.ShapeDtypeStruct((1, 8), jnp.float32),
        ],
    )(acc1, bc1, wc2, s2, d2)


def _sc2_body(edge_ref, t2_ref, ad2_ref, cvec_ref, out_ref,
              acc, cbuf,
              rb0, rb1, ab0, ab1, sb0, sb1,
              eidx, si0, si1, di0, di1, dS0, dS1, dSt,
              sg0, sg1, sa0, sa1, ss0, ss1):
    cid = lax.axis_index("c")
    sid = lax.axis_index("s")
    pltpu.sync_copy(cvec_ref, cbuf)
    cv = cbuf[...]
    scale = jnp.where(cid == 0, 1.0, 0.0)

    for j in range((_NCHUNK + 15) // 16):
        cidx = sid + 16 * j

        @pl.when(cidx < _NCHUNK)
        def _():
            rb = cidx * _NB
            pltpu.sync_copy(t2_ref.at[pl.ds(rb, _NB)], rb0.at[pl.ds(0, _NB)])
            pltpu.sync_copy(ad2_ref.at[pl.ds(rb, _NB)], ab0.at[pl.ds(0, _NB)])

            @plsc.parallel_loop(0, _NB, unroll=4)
            def _init_row(r):
                asv = rb0[r, pl.ds(0, 16)]
                adv = ab0[r, pl.ds(0, 16)]
                z = asv + adv
                al = jnp.where(z > 0, z, 0.2 * z) - cv
                exv = jnp.exp(al) * scale
                rb0[r, pl.ds(0, 16)] = exv
                ex0 = exv[0]
                for k in range(2):
                    c0 = 16 + 16 * k
                    rb0[r, pl.ds(c0, 16)] = ex0 * rb0[r, pl.ds(c0, 16)]
            pltpu.sync_copy(rb0.at[pl.ds(0, _NB)], acc.at[pl.ds(rb, _NB)])

    plsc.subcore_barrier()

    bufs = ((rb0, ab0, sb0, si0, di0, dS0, sg0, sa0, ss0),
            (rb1, ab1, sb1, si1, di1, dS1, sg1, sa1, ss1))
    tbase = cid * (E // 2) + sid * _ET2

    def _refill(q):
        pltpu.sync_copy(edge_ref.at[:, pl.ds(tbase + q * _IFILL, _IFILL)],
                        eidx)

    def _issue(i, bid):
        rbE, abE, sbuf, sidx, didx2, didxS, sg, sa, ss = bufs[i]
        m = _EB2 * lax.rem(bid, _IFILL // _EB2)
        for k in range(_EB2 // 16):
            sl = pl.ds(16 * k, 16)
            sidx[sl] = eidx[0, pl.ds(m + 16 * k, 16)]
            didx2[sl] = eidx[1, pl.ds(m + 16 * k, 16)]
        pltpu.async_copy(t2_ref.at[sidx], rbE, sg)
        pltpu.async_copy(ad2_ref.at[didx2], abE, sa)

    def _wait_gather(i):
        rbE, abE, sbuf, sidx, didx2, didxS, sg, sa, ss = bufs[i]
        pltpu.make_async_copy(t2_ref.at[sidx], rbE, sg).wait()
        pltpu.make_async_copy(ad2_ref.at[didx2], abE, sa).wait()

    def _wait_scatter(i):
        rbE, abE, sbuf, sidx, didx2, didxS, sg, sa, ss = bufs[i]
        pltpu.make_async_copy(sbuf, acc.at[didxS], ss).wait()

    def _compute_scatter(i):
        rbE, abE, sbuf, sidx, didx2, didxS, sg, sa, ss = bufs[i]
        for k in range(_EB2 // 16):
            sl = pl.ds(16 * k, 16)
            didxS[sl] = didx2[sl]

        @plsc.parallel_loop(0, _EB2, unroll=8)
        def _edge(e):
            asv = rbE[e, pl.ds(0, 16)]
            adv = abE[e, pl.ds(0, 16)]
            z = asv + adv
            al = jnp.where(z > 0, z, 0.2 * z) - cv
            exv = jnp.exp(al)
            sbuf[e, pl.ds(0, 16)] = exv
            ex0 = exv[0]
            for k in range(2):
                c0 = 16 + 16 * k
                sbuf[e, pl.ds(c0, 16)] = ex0 * rbE[e, pl.ds(c0, 16)]

        pltpu.async_copy(sbuf, acc.at[didxS], ss, add=True)

    _refill(0)
    _issue(0, 0)
    _issue(1, 1)

    def _pair(p, _):
        @pl.when(lax.rem(p + 1, (_IFILL // _EB2) // 2) == 0)
        def _():
            _refill((p + 1) // ((_IFILL // _EB2) // 2))

        _wait_gather(0)

        @pl.when(p > 0)
        def _():
            _wait_scatter(0)

        _compute_scatter(0)

        @pl.when(2 * p + 2 < _NF2)
        def _():
            _issue(0, 2 * p + 2)

        _wait_gather(1)

        @pl.when(p > 0)
        def _():
            _wait_scatter(1)

        _compute_scatter(1)

        @pl.when(2 * p + 3 < _NF2)
        def _():
            _issue(1, 2 * p + 3)

        return 0

    lax.fori_loop(0, _P2, _pair, 0)
    _wait_scatter(0)
    _wait_scatter(1)

    for k in range(_TL2 // 16):
        sl = pl.ds(16 * k, 16)
        si0[sl] = eidx[0, pl.ds(_TC2 + 16 * k, 16)]
        dSt[sl] = eidx[1, pl.ds(_TC2 + 16 * k, 16)]
    pltpu.async_copy(t2_ref.at[si0.at[pl.ds(0, _TL2)]],
                     rb0.at[pl.ds(0, _TL2)], sg0)
    pltpu.async_copy(ad2_ref.at[dSt], ab0.at[pl.ds(0, _TL2)], sa0)
    pltpu.make_async_copy(t2_ref.at[si0.at[pl.ds(0, _TL2)]],
                          rb0.at[pl.ds(0, _TL2)], sg0).wait()
    pltpu.make_async_copy(ad2_ref.at[dSt], ab0.at[pl.ds(0, _TL2)],
                          sa0).wait()

    def _tail_edge(e, _):
        asv = rb0[e, pl.ds(0, 16)]
        adv = ab0[e, pl.ds(0, 16)]
        z = asv + adv
        al = jnp.where(z > 0, z, 0.2 * z) - cv
        exv = jnp.exp(al)
        sb0[e, pl.ds(0, 16)] = exv
        ex0 = exv[0]
        for k in range(2):
            c0 = 16 + 16 * k
            sb0[e, pl.ds(c0, 16)] = ex0 * rb0[e, pl.ds(c0, 16)]
        return 0

    lax.fori_loop(0, _TL2, _tail_edge, 0)
    pltpu.async_copy(sb0.at[pl.ds(0, _TL2)], acc.at[dSt], ss0, add=True)
    pltpu.make_async_copy(sb0.at[pl.ds(0, _TL2)], acc.at[dSt], ss0).wait()
    plsc.subcore_barrier()

    for j in range((_NCHUNK + 15) // 16):
        cidx = sid + 16 * j

        @pl.when(cidx < _NCHUNK)
        def _():
            rb = cidx * _NB
            pltpu.sync_copy(acc.at[pl.ds(rb, _NB)], rb0.at[pl.ds(0, _NB)])
            pltpu.sync_copy(rb0.at[pl.ds(0, _NB)],
                            out_ref.at[cid, pl.ds(rb, _NB)])


def _sc2(edge_index, t2, ad2, cvec):
    mesh = plsc.VectorSubcoreMesh(core_axis_name="c", subcore_axis_name="s")
    f = pl.kernel(
        _sc2_body,
        out_type=jax.ShapeDtypeStruct((2, N, _G2W), jnp.float32),
        mesh=mesh,
        scratch_types=[
            pltpu.VMEM_SHARED((N, _G2W), jnp.float32),
            pltpu.VMEM((16,), jnp.float32),
            pltpu.VMEM((_EB2, _G2W), jnp.float32),
            pltpu.VMEM((_EB2, _G2W), jnp.float32),
            pltpu.VMEM((_EB2, 16), jnp.float32),
            pltpu.VMEM((_EB2, 16), jnp.float32),
            pltpu.VMEM((_EB2, _G2W), jnp.float32),
            pltpu.VMEM((_EB2, _G2W), jnp.float32),
            pltpu.VMEM((2, _IFILL), jnp.int32),
            pltpu.VMEM((_EB2,), jnp.int32),
            pltpu.VMEM((_EB2,), jnp.int32),
            pltpu.VMEM((_EB2,), jnp.int32),
            pltpu.VMEM((_EB2,), jnp.int32),
            pltpu.VMEM((_EB2,), jnp.int32),
            pltpu.VMEM((_EB2,), jnp.int32),
            pltpu.VMEM((_TL2,), jnp.int32),
            pltpu.SemaphoreType.DMA,
            pltpu.SemaphoreType.DMA,
            pltpu.SemaphoreType.DMA,
            pltpu.SemaphoreType.DMA,
            pltpu.SemaphoreType.DMA,
            pltpu.SemaphoreType.DMA,
        ],
        compiler_params=pltpu.CompilerParams(use_tc_tiling_on_sc=False),
    )
    return f(edge_index, t2, ad2, cvec)


def _dense3_body(acc_ref, bc2_ref, w2_ref, b2_ref, out_ref):
    a0 = acc_ref[0]
    a1 = acc_ref[1]
    msg = a0[:, 16:_G2W] + a1[:, 16:_G2W]
    den = jnp.broadcast_to(a0[:, 0:1] + a1[:, 0:1], (_BLK, HID))
    g2 = msg / (den + 1e-16) + bc2_ref[...][None, :]
    out_ref[...] = (
        jnp.dot(g2, w2_ref[...], preferred_element_type=jnp.float32)
        + b2_ref[...][None, :])


def _dense3(acc2, bc2, w2, b2):
    return pl.pallas_call(
        _dense3_body,
        grid=(N // _BLK,),
        in_specs=[
            pl.BlockSpec((2, _BLK, _G2W), lambda i: (0, i, 0)),
            pl.BlockSpec((HID,), lambda i: (0,)),
            pl.BlockSpec((HID, OUT_CH), lambda i: (0, 0)),
            pl.BlockSpec((OUT_CH,), lambda i: (0,)),
        ],
        out_specs=pl.BlockSpec((_BLK, OUT_CH), lambda i: (i, 0)),
        out_shape=jax.ShapeDtypeStruct((N, OUT_CH), jnp.float32),
    )(acc2, bc2, w2, b2)


def kernel(x, edge_index, W_lin1, b_lin1, Wc1, bc1, attc1_s, attc1_d,
           Wc2, bc2, attc2_s, attc2_d, W_lin2, b_lin2):
    eye = jnp.eye(HEADS, dtype=jnp.float32)
    s1 = (eye[:, None, :] * attc1_s.T[None, :, :]).reshape(HEADS * HID, HEADS)
    d1 = (eye[:, None, :] * attc1_d.T[None, :, :]).reshape(HEADS * HID, HEADS)
    s2 = jnp.pad(attc2_s.T, ((0, 0), (0, 7)))
    d2 = jnp.pad(attc2_d.T, ((0, 0), (0, 7)))

    epad = jnp.pad(edge_index, ((0, 0), (0, _EPAD - E)))

    g, adt, mxs, mxd = _dense1(x, W_lin1, b_lin1, Wc1, s1, d1)
    c1 = jnp.maximum(mxs[0] + mxd[0], 0.0)
    cvec1 = jnp.zeros((2, 16), jnp.float32).at[:, :4].set(c1.reshape(2, 4))
    acc1 = _sc1(epad, g.reshape(2 * N, _G1W),
                adt.reshape(2 * N, 16), cvec1)

    t2, ad2, mxs2, mxd2 = _dense2(acc1, bc1, Wc2, s2, d2)
    c2 = jnp.maximum(mxs2[0, 0] + mxd2[0, 0], 0.0)
    cvec2 = jnp.zeros((16,), jnp.float32).at[0].set(c2)
    acc2 = _sc2(epad, t2, ad2, cvec2)

    return _dense3(acc2, bc2, W_lin2, b_lin2)
